# baseline (speedup 1.0000x reference)
import functools

import jax
import jax.numpy as jnp
from jax import lax
from jax.experimental import pallas as pl
from jax.experimental.pallas import tpu as pltpu
from jax.experimental.pallas import tpu_sc as plsc

EPS = 1e-6
BLK = 256
BLKF = 128
NC, NS, L = 2, 16, 16


def _rot_perm(hd):
    h = hd // 2
    eye = jnp.eye(h, dtype=jnp.float32)
    z = jnp.zeros((h, h), jnp.float32)
    return jnp.block([[z, eye], [-eye, z]])


def _rms_body(x_ref, anw_ref, a_ref):
    xs = x_ref[...]
    a = xs * jax.lax.rsqrt(jnp.mean(xs * xs, axis=-1, keepdims=True) + EPS)
    a_ref[...] = (a * anw_ref[...]).astype(jnp.bfloat16)


def _head_proj(a, w, nw, p, c, s, *, hd):
    q = jnp.dot(a, w, preferred_element_type=jnp.float32)
    qn = q * jax.lax.rsqrt(jnp.mean(q * q, axis=-1, keepdims=True) + EPS) * nw
    qr = jnp.dot(qn.astype(jnp.bfloat16), p, preferred_element_type=jnp.float32)
    return (qn * c + qr * s).astype(jnp.bfloat16)


def _qproj_body(a_ref, w_ref, nw_ref, c_ref, s_ref, p_ref, o_ref, *, hd):
    o_ref[0] = _head_proj(a_ref[...], w_ref[0], nw_ref[...], p_ref[...],
                          c_ref[...], s_ref[...], hd=hd)


def _kvproj_body(a_ref, wk_ref, wv_ref, nw_ref, c_ref, s_ref, p_ref,
                 k_ref, v_ref, *, hd):
    a = a_ref[...]
    k_ref[0] = _head_proj(a, wk_ref[0], nw_ref[...], p_ref[...],
                          c_ref[...], s_ref[...], hd=hd)
    v_ref[0] = jnp.dot(a, wv_ref[0],
                       preferred_element_type=jnp.float32).astype(jnp.bfloat16)


def _attn_body(q_ref, k_ref, v_ref, o_ref, *, hd):
    s = jax.lax.dot_general(q_ref[0], k_ref[0],
                            (((1,), (1,)), ((), ())),
                            preferred_element_type=jnp.float32)
    s = s * (1.0 / (hd ** 0.5))
    m = jnp.max(s, axis=-1, keepdims=True)
    e = jnp.exp(s - m)
    p = e / jnp.sum(e, axis=-1, keepdims=True)
    o_ref[0] = jnp.dot(p.astype(jnp.bfloat16), v_ref[0],
                       preferred_element_type=jnp.float32).astype(jnp.bfloat16)


def _post_body(ctx_ref, wo_ref, x_ref, fnw_ref, rw_ref,
               x2_ref, m_ref, oa_ref, ob_ref, wa_ref, wb_ref, *, ne, nq):
    h = pl.program_id(1)

    @pl.when(h == 0)
    def _init():
        x2_ref[...] = x_ref[...]

    x2_ref[...] += jnp.dot(ctx_ref[0], wo_ref[0],
                           preferred_element_type=jnp.float32)

    @pl.when(h == nq - 1)
    def _router():
        x2 = x2_ref[...]
        mm = x2 * jax.lax.rsqrt(jnp.mean(x2 * x2, axis=-1, keepdims=True) + EPS)
        mm = mm * fnw_ref[...]
        m_ref[...] = mm.astype(jnp.bfloat16)
        logits = jnp.dot(mm, rw_ref[...], preferred_element_type=jnp.float32)
        mx = jnp.max(logits, axis=-1, keepdims=True)
        ex = jnp.exp(logits - mx)
        g = ex / jnp.sum(ex, axis=-1, keepdims=True)
        it = jax.lax.broadcasted_iota(jnp.int32, g.shape, 1)
        m1 = jnp.max(g, axis=-1, keepdims=True)
        i1 = jnp.min(jnp.where(g == m1, it, ne), axis=-1, keepdims=True)
        g2 = jnp.where(it == i1, -jnp.inf, g)
        m2 = jnp.max(g2, axis=-1, keepdims=True)
        i2 = jnp.min(jnp.where(g2 == m2, it, ne), axis=-1, keepdims=True)
        oa_ref[...] = (it == i1).astype(jnp.float32)
        ob_ref[...] = (it == i2).astype(jnp.float32)
        wa_ref[...] = m1[:, 0]
        wb_ref[...] = m2[:, 0]


def _route_body(oa_ref, ob_ref, ls_ref, posa_ref, posb_ref, be_ref,
                *, ne, nblk):
    oa = oa_ref[...]
    ob = ob_ref[...]
    t = oa.shape[0]
    oab = oa.astype(jnp.bfloat16)
    obb = ob.astype(jnp.bfloat16)
    ls = ls_ref[...]
    ra = jnp.dot(ls, oab, preferred_element_type=jnp.float32)
    rb = jnp.dot(ls, obb, preferred_element_type=jnp.float32)
    tot_a = jnp.sum(oa, axis=0, keepdims=True)
    cnt = tot_a + jnp.sum(ob, axis=0, keepdims=True)
    blocks = jnp.floor((cnt + (BLKF - 1)) * (1.0 / BLKF))
    eiota_r = jax.lax.broadcasted_iota(jnp.int32, (ne, ne), 0)
    eiota_c = jax.lax.broadcasted_iota(jnp.int32, (ne, ne), 1)
    m8 = (eiota_r < eiota_c).astype(jnp.float32)
    sblk = jnp.dot(blocks, m8, preferred_element_type=jnp.float32)
    spad = sblk * BLKF
    posa = jnp.sum(oa * (ra + spad), axis=1)
    posb = jnp.sum(ob * (rb + tot_a + spad), axis=1)
    posa_ref[...] = posa.astype(jnp.int32)
    posb_ref[...] = posb.astype(jnp.int32)
    cnt_t = jax.lax.dot_general(oa + ob, jnp.ones((t, 1), jnp.float32),
                                (((0,), (0,)), ((), ())),
                                preferred_element_type=jnp.float32)
    blocks_t = jnp.floor((cnt_t + (BLKF - 1)) * (1.0 / BLKF))
    m8l = (eiota_c < eiota_r).astype(jnp.float32)
    sblk_t = jnp.dot(m8l, blocks_t, preferred_element_type=jnp.float32)
    biota = jax.lax.broadcasted_iota(jnp.int32, (ne, nblk), 1).astype(jnp.float32)
    ge = (biota >= sblk_t).astype(jnp.float32)
    be = jnp.sum(ge, axis=0, keepdims=True) - 1.0
    total = jnp.sum(blocks_t)
    biota1 = jax.lax.broadcasted_iota(jnp.int32, (1, nblk), 1).astype(jnp.float32)
    be = jnp.where(biota1 < total, be, -1.0)
    be_ref[...] = be[0].astype(jnp.int32)


def _gffn_body(be_ref, xg_ref, wg_ref, wi_ref, woe_ref, y_ref):
    b = pl.program_id(0)

    @pl.when(be_ref[b] >= 0)
    def _compute():
        mb = xg_ref[...]
        g = jnp.dot(mb, wg_ref[0], preferred_element_type=jnp.float32)
        u = jnp.dot(mb, wi_ref[0], preferred_element_type=jnp.float32)
        h = (g * jax.nn.sigmoid(g) * u).astype(jnp.bfloat16)
        y_ref[...] = jnp.dot(h, woe_ref[0],
                             preferred_element_type=jnp.float32).astype(jnp.bfloat16)


def _sc_mesh():
    return plsc.VectorSubcoreMesh(core_axis_name="c", subcore_axis_name="s",
                                  num_cores=NC, num_subcores=NS)


def _sc_dispatch(pos_all, m, *, nrows, t, dim, nslots):
    dim2 = dim // 2
    m32 = jax.lax.bitcast_convert_type(m.reshape(t, dim2, 2), jnp.uint32)
    nw = NC * NS
    spt = nslots // nw

    @functools.partial(
        pl.kernel,
        out_type=jax.ShapeDtypeStruct((nrows, dim2), jnp.uint32),
        mesh=_sc_mesh(),
        scratch_types=[pltpu.VMEM((spt,), jnp.int32),
                       pltpu.VMEM((spt, dim2), jnp.uint32),
                       pltpu.SemaphoreType.DMA],
        compiler_params=pltpu.CompilerParams(needs_layout_passes=False),
    )
    def _dispatch(pos_hbm, m_hbm, xg_hbm, idx_v, rows_v, sem):
        wid = lax.axis_index("s") * NC + lax.axis_index("c")
        base = wid * spt
        tok0 = base % t
        pltpu.sync_copy(pos_hbm.at[pl.ds(base, spt)], idx_v)
        pltpu.sync_copy(m_hbm.at[pl.ds(tok0, spt)], rows_v)
        pltpu.async_copy(rows_v, xg_hbm.at[idx_v], sem).wait()

    return _dispatch(pos_all, m32)


def _sc_combine(pos_all, y, *, dim, nslots):
    nry = y.shape[0]
    dim2 = dim // 2
    y32 = jax.lax.bitcast_convert_type(y.reshape(nry, dim2, 2), jnp.uint32)
    nw = NC * NS
    spt = nslots // nw

    @functools.partial(
        pl.kernel,
        out_type=jax.ShapeDtypeStruct((nslots, dim2), jnp.uint32),
        mesh=_sc_mesh(),
        scratch_types=[pltpu.VMEM((spt,), jnp.int32),
                       pltpu.VMEM((spt, dim2), jnp.uint32),
                       pltpu.SemaphoreType.DMA],
        compiler_params=pltpu.CompilerParams(needs_layout_passes=False),
    )
    def _combine(pos_hbm, y_hbm, yg_hbm, idx_v, rows_v, sem):
        wid = lax.axis_index("s") * NC + lax.axis_index("c")
        base = wid * spt
        pltpu.sync_copy(pos_hbm.at[pl.ds(base, spt)], idx_v)
        pltpu.async_copy(y_hbm.at[idx_v], rows_v, sem).wait()
        pltpu.sync_copy(rows_v, yg_hbm.at[pl.ds(base, spt)])

    yg32 = _combine(pos_all, y32)
    return jax.lax.bitcast_convert_type(yg32, jnp.bfloat16).reshape(nslots, dim)


def _final_body(x2_ref, ya_ref, yb_ref, wa_ref, wb_ref, o_ref):
    wa = jnp.reshape(wa_ref[...], (-1, 1))
    wb = jnp.reshape(wb_ref[...], (-1, 1))
    o_ref[...] = (x2_ref[...] + wa * ya_ref[...].astype(jnp.float32)
                  + wb * yb_ref[...].astype(jnp.float32))


def kernel(x, attn_norm_w, Wq, Wk, Wv, Wo, q_norm_w, k_norm_w, ffn_norm_w,
           Wi, Wg, Woe, router_w, cos, sin):
    b, t, dim = x.shape
    nq = Wq.shape[1] // cos.shape[1]
    nkv = Wk.shape[1] // cos.shape[1]
    hd = cos.shape[1]
    ne, _, hid = Wi.shape
    blk = min(BLK, t)
    nt = t // blk
    nslots = 2 * t
    nblk = nslots // BLKF + ne
    nrows = nblk * BLKF

    x2d = x.reshape(t, dim)
    bf = jnp.bfloat16
    f32 = jnp.float32
    wo3 = Wo.reshape(nq, hd, dim).astype(bf)
    wq3 = Wq.reshape(dim, nq, hd).transpose(1, 0, 2).astype(bf)
    wk3 = Wk.reshape(dim, nkv, hd).transpose(1, 0, 2).astype(bf)
    wv3 = Wv.reshape(dim, nkv, hd).transpose(1, 0, 2).astype(bf)
    wi_b, wg_b, woe_b = (w.astype(bf) for w in (Wi, Wg, Woe))

    p64 = _rot_perm(hd).astype(bf)
    qnw = q_norm_w.reshape(1, hd)
    knw = k_norm_w.reshape(1, hd)
    anw = attn_norm_w.reshape(1, dim)
    fnw = ffn_norm_w.reshape(1, dim)
    tio_r = jax.lax.broadcasted_iota(jnp.int32, (t, t), 0)
    tio_c = jax.lax.broadcasted_iota(jnp.int32, (t, t), 1)
    ls2048 = (tio_c < tio_r).astype(bf)

    full = lambda shape: pl.BlockSpec(shape, lambda *_: (0,) * len(shape))
    rowblk = lambda w: pl.BlockSpec((blk, w), lambda i: (i, 0))

    a = pl.pallas_call(
        _rms_body,
        grid=(nt,),
        in_specs=[rowblk(dim), full((1, dim))],
        out_specs=rowblk(dim),
        out_shape=jax.ShapeDtypeStruct((t, dim), bf),
    )(x2d, anw)

    hblk = lambda: pl.BlockSpec((1, blk, hd), lambda h, i: (h, i, 0))
    tcs = lambda: pl.BlockSpec((blk, hd), lambda h, i: (i, 0))
    q3 = pl.pallas_call(
        functools.partial(_qproj_body, hd=hd),
        grid=(nq, nt),
        in_specs=[pl.BlockSpec((blk, dim), lambda h, i: (i, 0)),
                  pl.BlockSpec((1, dim, hd), lambda h, i: (h, 0, 0)),
                  full((1, hd)), tcs(), tcs(), full((hd, hd))],
        out_specs=hblk(),
        out_shape=jax.ShapeDtypeStruct((nq, t, hd), bf),
    )(a, wq3, qnw, cos, sin, p64)

    k3, v3 = pl.pallas_call(
        functools.partial(_kvproj_body, hd=hd),
        grid=(nkv, nt),
        in_specs=[pl.BlockSpec((blk, dim), lambda h, i: (i, 0)),
                  pl.BlockSpec((1, dim, hd), lambda h, i: (h, 0, 0)),
                  pl.BlockSpec((1, dim, hd), lambda h, i: (h, 0, 0)),
                  full((1, hd)), tcs(), tcs(), full((hd, hd))],
        out_specs=[pl.BlockSpec((1, blk, hd), lambda h, i: (h, i, 0)),
                   pl.BlockSpec((1, blk, hd), lambda h, i: (h, i, 0))],
        out_shape=[jax.ShapeDtypeStruct((nkv, t, hd), bf),
                   jax.ShapeDtypeStruct((nkv, t, hd), bf)],
    )(a, wk3, wv3, knw, cos, sin, p64)

    rep = nq // nkv
    ctx3 = pl.pallas_call(
        functools.partial(_attn_body, hd=hd),
        grid=(nq, nt),
        in_specs=[
            pl.BlockSpec((1, blk, hd), lambda h, i: (h, i, 0)),
            pl.BlockSpec((1, t, hd), lambda h, i: (h // rep, 0, 0)),
            pl.BlockSpec((1, t, hd), lambda h, i: (h // rep, 0, 0)),
        ],
        out_specs=pl.BlockSpec((1, blk, hd), lambda h, i: (h, i, 0)),
        out_shape=jax.ShapeDtypeStruct((nq, t, hd), bf),
    )(q3, k3, v3)

    x2, m, oa, ob, wa, wb = pl.pallas_call(
        functools.partial(_post_body, ne=ne, nq=nq),
        grid=(nt, nq),
        in_specs=[pl.BlockSpec((1, blk, hd), lambda i, h: (h, i, 0)),
                  pl.BlockSpec((1, hd, dim), lambda i, h: (h, 0, 0)),
                  pl.BlockSpec((blk, dim), lambda i, h: (i, 0)),
                  full((1, dim)), full((dim, ne))],
        out_specs=[pl.BlockSpec((blk, dim), lambda i, h: (i, 0)),
                   pl.BlockSpec((blk, dim), lambda i, h: (i, 0)),
                   pl.BlockSpec((blk, ne), lambda i, h: (i, 0)),
                   pl.BlockSpec((blk, ne), lambda i, h: (i, 0)),
                   pl.BlockSpec((blk,), lambda i, h: (i,)),
                   pl.BlockSpec((blk,), lambda i, h: (i,))],
        out_shape=[
            jax.ShapeDtypeStruct((t, dim), f32),
            jax.ShapeDtypeStruct((t, dim), bf),
            jax.ShapeDtypeStruct((t, ne), f32),
            jax.ShapeDtypeStruct((t, ne), f32),
            jax.ShapeDtypeStruct((t,), f32),
            jax.ShapeDtypeStruct((t,), f32),
        ],
        compiler_params=pltpu.CompilerParams(
            dimension_semantics=("parallel", "arbitrary")),
    )(ctx3, wo3, x2d, fnw, router_w)

    posa, posb, be = pl.pallas_call(
        functools.partial(_route_body, ne=ne, nblk=nblk),
        grid=(1,),
        in_specs=[full((t, ne)), full((t, ne)), full((t, t))],
        out_specs=[pl.BlockSpec((t,), lambda i: (0,)),
                   pl.BlockSpec((t,), lambda i: (0,)),
                   pl.BlockSpec((nblk,), lambda i: (0,))],
        out_shape=[
            jax.ShapeDtypeStruct((t,), jnp.int32),
            jax.ShapeDtypeStruct((t,), jnp.int32),
            jax.ShapeDtypeStruct((nblk,), jnp.int32),
        ],
    )(oa, ob, ls2048)

    pos_all = jnp.concatenate([posa, posb])
    xg32 = _sc_dispatch(pos_all, m, nrows=nrows, t=t, dim=dim, nslots=nslots)
    xg = jax.lax.bitcast_convert_type(xg32, bf).reshape(nrows, dim)

    y = pl.pallas_call(
        _gffn_body,
        grid_spec=pltpu.PrefetchScalarGridSpec(
            num_scalar_prefetch=1,
            grid=(nblk,),
            in_specs=[
                pl.BlockSpec((BLKF, dim), lambda bi, be_s: (bi, 0)),
                pl.BlockSpec((1, dim, hid),
                             lambda bi, be_s: (jnp.maximum(be_s[bi], 0), 0, 0)),
                pl.BlockSpec((1, dim, hid),
                             lambda bi, be_s: (jnp.maximum(be_s[bi], 0), 0, 0)),
                pl.BlockSpec((1, hid, dim),
                             lambda bi, be_s: (jnp.maximum(be_s[bi], 0), 0, 0)),
            ],
            out_specs=pl.BlockSpec((BLKF, dim), lambda bi, be_s: (bi, 0)),
        ),
        out_shape=jax.ShapeDtypeStruct((nrows, dim), bf),
        compiler_params=pltpu.CompilerParams(
            dimension_semantics=("arbitrary",)),
    )(be, xg, wg_b, wi_b, woe_b)

    yg = _sc_combine(pos_all, y, dim=dim, nslots=nslots)
    ya, yb = yg[:t], yg[t:]

    out = pl.pallas_call(
        _final_body,
        grid=(nt,),
        in_specs=[rowblk(dim), rowblk(dim), rowblk(dim),
                  pl.BlockSpec((blk,), lambda i: (i,)),
                  pl.BlockSpec((blk,), lambda i: (i,))],
        out_specs=rowblk(dim),
        out_shape=jax.ShapeDtypeStruct((t, dim), f32),
    )(x2, ya, yb, wa, wb)

    return out.reshape(b, t, dim)


# speedup vs baseline: 2.1394x; 2.1394x over previous
import functools

import jax
import jax.numpy as jnp
from jax import lax
from jax.experimental import pallas as pl
from jax.experimental.pallas import tpu as pltpu
from jax.experimental.pallas import tpu_sc as plsc

EPS = 1e-6
BLK = 256
BLKF = 128
NC, NS, L = 2, 16, 16


def _rot_perm(hd):
    h = hd // 2
    eye = jnp.eye(h, dtype=jnp.float32)
    z = jnp.zeros((h, h), jnp.float32)
    return jnp.block([[z, eye], [-eye, z]])


def _prelude_body(x_ref, anw_ref, wq_ref, wk_ref, wv_ref, qnw_ref, knw_ref,
                  cq_ref, sq_ref, ck_ref, sk_ref, pq_ref, pk_ref,
                  hq_ref, hqt_ref, hk_ref, hkt_ref,
                  q_ref, k_ref, v_ref, *, hd):
    xs = x_ref[...]
    a = xs * jax.lax.rsqrt(jnp.mean(xs * xs, axis=-1, keepdims=True) + EPS)
    a = (a * anw_ref[...]).astype(jnp.bfloat16)

    def qk_path(w_ref, nw_ref, h_ref, ht_ref, p_ref, c_ref, s_ref):
        q = jnp.dot(a, w_ref[...], preferred_element_type=jnp.float32)
        ss = jnp.dot(q * q, h_ref[...], preferred_element_type=jnp.float32)
        rs = jax.lax.rsqrt(ss / hd + EPS)
        qn = q * jnp.dot(rs, ht_ref[...], preferred_element_type=jnp.float32)
        qn = qn * nw_ref[...]
        qr = jnp.dot(qn.astype(jnp.bfloat16), p_ref[...],
                     preferred_element_type=jnp.float32)
        return (qn * c_ref[...] + qr * s_ref[...]).astype(jnp.bfloat16)

    q_ref[...] = qk_path(wq_ref, qnw_ref, hq_ref, hqt_ref, pq_ref, cq_ref, sq_ref)
    k_ref[...] = qk_path(wk_ref, knw_ref, hk_ref, hkt_ref, pk_ref, ck_ref, sk_ref)
    v_ref[...] = jnp.dot(a, wv_ref[...],
                         preferred_element_type=jnp.float32).astype(jnp.bfloat16)


def _attn_body(q_ref, k_ref, v_ref, o_ref, *, hd, nq, rep):
    qs = q_ref[...]
    ks = k_ref[...]
    vs = v_ref[...]
    outs = []
    for h in range(nq):
        g = h // rep
        q = qs[:, h * hd:(h + 1) * hd]
        k = ks[:, g * hd:(g + 1) * hd]
        v = vs[:, g * hd:(g + 1) * hd]
        s = jax.lax.dot_general(q, k, (((1,), (1,)), ((), ())),
                                preferred_element_type=jnp.float32)
        s = s * (1.0 / (hd ** 0.5))
        m = jnp.max(s, axis=-1, keepdims=True)
        e = jnp.exp(s - m)
        p = e / jnp.sum(e, axis=-1, keepdims=True)
        outs.append(jnp.dot(p.astype(jnp.bfloat16), v,
                            preferred_element_type=jnp.float32))
    o_ref[...] = jnp.concatenate(outs, axis=1).astype(jnp.bfloat16)


def _post_body(ctx_ref, wo_ref, x_ref, fnw_ref, rw_ref,
               x2_ref, m_ref, oa_ref, ob_ref, wa_ref, wb_ref, *, ne):
    x2 = x_ref[...] + jnp.dot(ctx_ref[...], wo_ref[...],
                              preferred_element_type=jnp.float32)
    x2_ref[...] = x2
    mm = x2 * jax.lax.rsqrt(jnp.mean(x2 * x2, axis=-1, keepdims=True) + EPS)
    mm = mm * fnw_ref[...]
    m_ref[...] = mm
    logits = jnp.dot(mm, rw_ref[...], preferred_element_type=jnp.float32)
    mx = jnp.max(logits, axis=-1, keepdims=True)
    ex = jnp.exp(logits - mx)
    g = ex / jnp.sum(ex, axis=-1, keepdims=True)
    it = jax.lax.broadcasted_iota(jnp.int32, g.shape, 1)
    m1 = jnp.max(g, axis=-1, keepdims=True)
    i1 = jnp.min(jnp.where(g == m1, it, ne), axis=-1, keepdims=True)
    g2 = jnp.where(it == i1, -jnp.inf, g)
    m2 = jnp.max(g2, axis=-1, keepdims=True)
    i2 = jnp.min(jnp.where(g2 == m2, it, ne), axis=-1, keepdims=True)
    oa_ref[...] = (it == i1).astype(jnp.float32)
    ob_ref[...] = (it == i2).astype(jnp.float32)
    wa_ref[...] = m1[:, 0]
    wb_ref[...] = m2[:, 0]


def _route_body(oa_ref, ob_ref, ls_ref, posa_ref, posb_ref, be_ref,
                *, ne, nblk):
    oa = oa_ref[...]
    ob = ob_ref[...]
    t = oa.shape[0]
    oab = oa.astype(jnp.bfloat16)
    obb = ob.astype(jnp.bfloat16)
    ls = ls_ref[...]
    ra = jnp.dot(ls, oab, preferred_element_type=jnp.float32)
    rb = jnp.dot(ls, obb, preferred_element_type=jnp.float32)
    tot_a = jnp.sum(oa, axis=0, keepdims=True)
    cnt = tot_a + jnp.sum(ob, axis=0, keepdims=True)
    blocks = jnp.floor((cnt + (BLKF - 1)) * (1.0 / BLKF))
    eiota_r = jax.lax.broadcasted_iota(jnp.int32, (ne, ne), 0)
    eiota_c = jax.lax.broadcasted_iota(jnp.int32, (ne, ne), 1)
    m8 = (eiota_r < eiota_c).astype(jnp.float32)
    sblk = jnp.dot(blocks, m8, preferred_element_type=jnp.float32)
    spad = sblk * BLKF
    posa = jnp.sum(oa * (ra + spad), axis=1)
    posb = jnp.sum(ob * (rb + tot_a + spad), axis=1)
    posa_ref[...] = posa.astype(jnp.int32)
    posb_ref[...] = posb.astype(jnp.int32)
    cnt_t = jax.lax.dot_general(oa + ob, jnp.ones((t, 1), jnp.float32),
                                (((0,), (0,)), ((), ())),
                                preferred_element_type=jnp.float32)
    blocks_t = jnp.floor((cnt_t + (BLKF - 1)) * (1.0 / BLKF))
    m8l = (eiota_c < eiota_r).astype(jnp.float32)
    sblk_t = jnp.dot(m8l, blocks_t, preferred_element_type=jnp.float32)
    biota = jax.lax.broadcasted_iota(jnp.int32, (ne, nblk), 1).astype(jnp.float32)
    ge = (biota >= sblk_t).astype(jnp.float32)
    be = jnp.sum(ge, axis=0, keepdims=True) - 1.0
    total = jnp.sum(blocks_t)
    biota1 = jax.lax.broadcasted_iota(jnp.int32, (1, nblk), 1).astype(jnp.float32)
    be = jnp.where(biota1 < total, be, -1.0)
    be_ref[...] = be[0].astype(jnp.int32)


def _gffn_body(be_ref, xg_ref, wg_ref, wi_ref, woe_ref, y_ref):
    b = pl.program_id(0)

    @pl.when(be_ref[b] >= 0)
    def _compute():
        mb = xg_ref[...].astype(jnp.bfloat16)
        g = jnp.dot(mb, wg_ref[0], preferred_element_type=jnp.float32)
        u = jnp.dot(mb, wi_ref[0], preferred_element_type=jnp.float32)
        h = (g * jax.nn.sigmoid(g) * u).astype(jnp.bfloat16)
        y_ref[...] = jnp.dot(h, woe_ref[0], preferred_element_type=jnp.float32)


def _sc_mesh():
    return plsc.VectorSubcoreMesh(core_axis_name="c", subcore_axis_name="s",
                                  num_cores=NC, num_subcores=NS)


def _sc_dispatch(pos_all, m, *, nrows, t, dim, nslots):
    nw = NC * NS
    spt = nslots // nw
    ch = spt // 2

    @functools.partial(
        pl.kernel,
        out_type=jax.ShapeDtypeStruct((nrows, dim), jnp.float32),
        mesh=_sc_mesh(),
        scratch_types=[pltpu.VMEM((ch,), jnp.int32),
                       pltpu.VMEM((ch,), jnp.int32),
                       pltpu.VMEM((ch, dim), jnp.float32),
                       pltpu.SemaphoreType.DMA],
        compiler_params=pltpu.CompilerParams(needs_layout_passes=False),
    )
    def _dispatch(pos_hbm, m_hbm, xg_hbm, idx_a, idx_b, rows_v, sem):
        wid = lax.axis_index("s") * NC + lax.axis_index("c")
        base = wid * spt
        tok0 = base % t
        pltpu.sync_copy(pos_hbm.at[pl.ds(base, ch)], idx_a)
        pltpu.sync_copy(pos_hbm.at[pl.ds(base + ch, ch)], idx_b)
        pltpu.sync_copy(m_hbm.at[pl.ds(tok0, ch)], rows_v)
        pltpu.async_copy(rows_v, xg_hbm.at[idx_a], sem).wait()
        pltpu.sync_copy(m_hbm.at[pl.ds(tok0 + ch, ch)], rows_v)
        pltpu.async_copy(rows_v, xg_hbm.at[idx_b], sem).wait()

    return _dispatch(pos_all, m)


def _sc_combine(pos_all, y, *, dim, nslots):
    nw = NC * NS
    spt = nslots // nw
    ch = spt // 2

    @functools.partial(
        pl.kernel,
        out_type=jax.ShapeDtypeStruct((nslots, dim), jnp.float32),
        mesh=_sc_mesh(),
        scratch_types=[pltpu.VMEM((ch,), jnp.int32),
                       pltpu.VMEM((ch,), jnp.int32),
                       pltpu.VMEM((ch, dim), jnp.float32),
                       pltpu.SemaphoreType.DMA],
        compiler_params=pltpu.CompilerParams(needs_layout_passes=False),
    )
    def _combine(pos_hbm, y_hbm, yg_hbm, idx_a, idx_b, rows_v, sem):
        wid = lax.axis_index("s") * NC + lax.axis_index("c")
        base = wid * spt
        pltpu.sync_copy(pos_hbm.at[pl.ds(base, ch)], idx_a)
        pltpu.sync_copy(pos_hbm.at[pl.ds(base + ch, ch)], idx_b)
        pltpu.async_copy(y_hbm.at[idx_a], rows_v, sem).wait()
        pltpu.sync_copy(rows_v, yg_hbm.at[pl.ds(base, ch)])
        pltpu.async_copy(y_hbm.at[idx_b], rows_v, sem).wait()
        pltpu.sync_copy(rows_v, yg_hbm.at[pl.ds(base + ch, ch)])

    return _combine(pos_all, y)


def _final_body(x2_ref, ya_ref, yb_ref, wa_ref, wb_ref, o_ref):
    wa = jnp.reshape(wa_ref[...], (-1, 1))
    wb = jnp.reshape(wb_ref[...], (-1, 1))
    o_ref[...] = x2_ref[...] + wa * ya_ref[...] + wb * yb_ref[...]


def kernel(x, attn_norm_w, Wq, Wk, Wv, Wo, q_norm_w, k_norm_w, ffn_norm_w,
           Wi, Wg, Woe, router_w, cos, sin):
    b, t, dim = x.shape
    nq = Wq.shape[1] // cos.shape[1]
    nkv = Wk.shape[1] // cos.shape[1]
    hd = cos.shape[1]
    ne, _, hid = Wi.shape
    blk = min(BLK, t)
    nt = t // blk
    nslots = 2 * t
    nblk = nslots // BLKF + ne
    nrows = nblk * BLKF

    x2d = x.reshape(t, dim)
    bf = jnp.bfloat16
    f32 = jnp.float32
    wq_b, wk_b, wv_b, wo_b = (w.astype(bf) for w in (Wq, Wk, Wv, Wo))
    wi_b, wg_b, woe_b = (w.astype(bf) for w in (Wi, Wg, Woe))

    p64 = _rot_perm(hd)
    pq = jnp.kron(jnp.eye(nq, dtype=f32), p64).astype(bf)
    pk = jnp.kron(jnp.eye(nkv, dtype=f32), p64).astype(bf)
    hq = jnp.kron(jnp.eye(nq, dtype=f32), jnp.ones((hd, 1), f32))
    hk = jnp.kron(jnp.eye(nkv, dtype=f32), jnp.ones((hd, 1), f32))
    cq = jnp.tile(cos, (1, nq))
    sq = jnp.tile(sin, (1, nq))
    ck = jnp.tile(cos, (1, nkv))
    sk = jnp.tile(sin, (1, nkv))
    qnw = jnp.tile(q_norm_w, (nq,)).reshape(1, nq * hd)
    knw = jnp.tile(k_norm_w, (nkv,)).reshape(1, nkv * hd)
    anw = attn_norm_w.reshape(1, dim)
    fnw = ffn_norm_w.reshape(1, dim)
    tio_r = jax.lax.broadcasted_iota(jnp.int32, (t, t), 0)
    tio_c = jax.lax.broadcasted_iota(jnp.int32, (t, t), 1)
    ls2048 = (tio_c < tio_r).astype(bf)

    dq, dkv = nq * hd, nkv * hd
    rep = nq // nkv

    full = lambda shape: pl.BlockSpec(shape, lambda *_: (0,) * len(shape))
    rowblk = lambda w: pl.BlockSpec((blk, w), lambda i: (i, 0))

    q, k, v = pl.pallas_call(
        functools.partial(_prelude_body, hd=hd),
        grid=(nt,),
        in_specs=[
            rowblk(dim), full((1, dim)), full((dim, dq)), full((dim, dkv)),
            full((dim, dkv)), full((1, dq)), full((1, dkv)),
            rowblk(dq), rowblk(dq), rowblk(dkv), rowblk(dkv),
            full((dq, dq)), full((dkv, dkv)),
            full((dq, nq)), full((nq, dq)), full((dkv, nkv)), full((nkv, dkv)),
        ],
        out_specs=[rowblk(dq), rowblk(dkv), rowblk(dkv)],
        out_shape=[
            jax.ShapeDtypeStruct((t, dq), bf),
            jax.ShapeDtypeStruct((t, dkv), bf),
            jax.ShapeDtypeStruct((t, dkv), bf),
        ],
    )(x2d, anw, wq_b, wk_b, wv_b, qnw, knw, cq, sq, ck, sk,
      pq, pk, hq, hq.T, hk, hk.T)

    ctx = pl.pallas_call(
        functools.partial(_attn_body, hd=hd, nq=nq, rep=rep),
        grid=(nt,),
        in_specs=[rowblk(dq), full((t, dkv)), full((t, dkv))],
        out_specs=rowblk(dq),
        out_shape=jax.ShapeDtypeStruct((t, dq), bf),
    )(q, k, v)

    x2, m, oa, ob, wa, wb = pl.pallas_call(
        functools.partial(_post_body, ne=ne),
        grid=(nt,),
        in_specs=[rowblk(dq), full((dq, dim)), rowblk(dim), full((1, dim)),
                  full((dim, ne))],
        out_specs=[rowblk(dim), rowblk(dim), rowblk(ne), rowblk(ne),
                   pl.BlockSpec((blk,), lambda i: (i,)),
                   pl.BlockSpec((blk,), lambda i: (i,))],
        out_shape=[
            jax.ShapeDtypeStruct((t, dim), f32),
            jax.ShapeDtypeStruct((t, dim), f32),
            jax.ShapeDtypeStruct((t, ne), f32),
            jax.ShapeDtypeStruct((t, ne), f32),
            jax.ShapeDtypeStruct((t,), f32),
            jax.ShapeDtypeStruct((t,), f32),
        ],
    )(ctx, wo_b, x2d, fnw, router_w)

    posa, posb, be = pl.pallas_call(
        functools.partial(_route_body, ne=ne, nblk=nblk),
        grid=(1,),
        in_specs=[full((t, ne)), full((t, ne)), full((t, t))],
        out_specs=[pl.BlockSpec((t,), lambda i: (0,)),
                   pl.BlockSpec((t,), lambda i: (0,)),
                   pl.BlockSpec((nblk,), lambda i: (0,))],
        out_shape=[
            jax.ShapeDtypeStruct((t,), jnp.int32),
            jax.ShapeDtypeStruct((t,), jnp.int32),
            jax.ShapeDtypeStruct((nblk,), jnp.int32),
        ],
    )(oa, ob, ls2048)

    pos_all = jnp.concatenate([posa, posb])
    xg = _sc_dispatch(pos_all, m, nrows=nrows, t=t, dim=dim, nslots=nslots)

    y = pl.pallas_call(
        _gffn_body,
        grid_spec=pltpu.PrefetchScalarGridSpec(
            num_scalar_prefetch=1,
            grid=(nblk,),
            in_specs=[
                pl.BlockSpec((BLKF, dim), lambda bi, be_s: (bi, 0)),
                pl.BlockSpec((1, dim, hid),
                             lambda bi, be_s: (jnp.maximum(be_s[bi], 0), 0, 0)),
                pl.BlockSpec((1, dim, hid),
                             lambda bi, be_s: (jnp.maximum(be_s[bi], 0), 0, 0)),
                pl.BlockSpec((1, hid, dim),
                             lambda bi, be_s: (jnp.maximum(be_s[bi], 0), 0, 0)),
            ],
            out_specs=pl.BlockSpec((BLKF, dim), lambda bi, be_s: (bi, 0)),
        ),
        out_shape=jax.ShapeDtypeStruct((nrows, dim), f32),
        compiler_params=pltpu.CompilerParams(
            dimension_semantics=("arbitrary",)),
    )(be, xg, wg_b, wi_b, woe_b)

    yg = _sc_combine(pos_all, y, dim=dim, nslots=nslots)
    ya, yb = yg[:t], yg[t:]

    out = pl.pallas_call(
        _final_body,
        grid=(nt,),
        in_specs=[rowblk(dim), rowblk(dim), rowblk(dim),
                  pl.BlockSpec((blk,), lambda i: (i,)),
                  pl.BlockSpec((blk,), lambda i: (i,))],
        out_specs=rowblk(dim),
        out_shape=jax.ShapeDtypeStruct((t, dim), f32),
    )(x2, ya, yb, wa, wb)

    return out.reshape(b, t, dim)


# speedup vs baseline: 2.6335x; 1.2310x over previous
import functools

import jax
import jax.numpy as jnp
import numpy as np
from jax import lax
from jax.experimental import pallas as pl
from jax.experimental.pallas import tpu as pltpu
from jax.experimental.pallas import tpu_sc as plsc

EPS = 1e-6
BLK = 256
BLKF = 128
NC, NS, L = 2, 16, 16


def _rot_perm(hd):
    h = hd // 2
    eye = np.eye(h, dtype=np.float32)
    z = np.zeros((h, h), np.float32)
    return np.block([[z, eye], [-eye, z]])


def _prelude_body(x_ref, anw_ref, wq_ref, wk_ref, wv_ref, qnw_ref, knw_ref,
                  c_ref, s_ref, pq_ref, pk_ref, hq_ref, hqt_ref,
                  hk_ref, hkt_ref, q_ref, k_ref, v_ref, *, hd, nq, nkv):
    xs = x_ref[...]
    a = xs * jax.lax.rsqrt(jnp.mean(xs * xs, axis=-1, keepdims=True) + EPS)
    a = (a * anw_ref[...]).astype(jnp.bfloat16)
    cb = c_ref[...]
    sb = s_ref[...]

    def qk_path(w_ref, nw_ref, p_ref, h_ref, ht_ref, nh):
        q = jnp.dot(a, w_ref[...].astype(jnp.bfloat16),
                    preferred_element_type=jnp.float32)
        ss = jnp.dot(q * q, h_ref[...], preferred_element_type=jnp.float32)
        rs = jax.lax.rsqrt(ss / hd + EPS)
        qn = q * jnp.dot(rs, ht_ref[...], preferred_element_type=jnp.float32)
        qn = qn * nw_ref[...]
        qr = jnp.dot(qn.astype(jnp.bfloat16), p_ref[...],
                     preferred_element_type=jnp.float32)
        c = jnp.concatenate([cb] * nh, axis=1)
        s = jnp.concatenate([sb] * nh, axis=1)
        return (qn * c + qr * s).astype(jnp.bfloat16)

    q_ref[...] = qk_path(wq_ref, qnw_ref, pq_ref, hq_ref, hqt_ref, nq)
    k_ref[...] = qk_path(wk_ref, knw_ref, pk_ref, hk_ref, hkt_ref, nkv)
    v_ref[...] = jnp.dot(a, wv_ref[...].astype(jnp.bfloat16),
                         preferred_element_type=jnp.float32).astype(jnp.bfloat16)


def _attn_body(q_ref, k_ref, v_ref, o_ref, *, hd, nq, rep):
    qs = q_ref[...]
    ks = k_ref[...]
    vs = v_ref[...]
    outs = []
    for h in range(nq):
        g = h // rep
        q = qs[:, h * hd:(h + 1) * hd]
        k = ks[:, g * hd:(g + 1) * hd]
        v = vs[:, g * hd:(g + 1) * hd]
        s = jax.lax.dot_general(q, k, (((1,), (1,)), ((), ())),
                                preferred_element_type=jnp.float32)
        s = s * (1.0 / (hd ** 0.5))
        m = jnp.max(s, axis=-1, keepdims=True)
        e = jnp.exp(s - m)
        p = e / jnp.sum(e, axis=-1, keepdims=True)
        outs.append(jnp.dot(p.astype(jnp.bfloat16), v,
                            preferred_element_type=jnp.float32))
    o_ref[...] = jnp.concatenate(outs, axis=1).astype(jnp.bfloat16)


def _post_body(ctx_ref, wo_ref, x_ref, fnw_ref, rw_ref,
               x2_ref, m_ref, oa_ref, ob_ref, wa_ref, wb_ref, *, ne):
    x2 = x_ref[...] + jnp.dot(ctx_ref[...], wo_ref[...].astype(jnp.bfloat16),
                              preferred_element_type=jnp.float32)
    x2_ref[...] = x2
    mm = x2 * jax.lax.rsqrt(jnp.mean(x2 * x2, axis=-1, keepdims=True) + EPS)
    mm = mm * fnw_ref[...]
    m_ref[...] = mm
    logits = jnp.dot(mm, rw_ref[...], preferred_element_type=jnp.float32)
    mx = jnp.max(logits, axis=-1, keepdims=True)
    ex = jnp.exp(logits - mx)
    g = ex / jnp.sum(ex, axis=-1, keepdims=True)
    it = jax.lax.broadcasted_iota(jnp.int32, g.shape, 1)
    m1 = jnp.max(g, axis=-1, keepdims=True)
    i1 = jnp.min(jnp.where(g == m1, it, ne), axis=-1, keepdims=True)
    g2 = jnp.where(it == i1, -jnp.inf, g)
    m2 = jnp.max(g2, axis=-1, keepdims=True)
    i2 = jnp.min(jnp.where(g2 == m2, it, ne), axis=-1, keepdims=True)
    oa_ref[...] = (it == i1).astype(jnp.float32)
    ob_ref[...] = (it == i2).astype(jnp.float32)
    wa_ref[...] = m1[:, 0]
    wb_ref[...] = m2[:, 0]


def _route_body(oa_ref, ob_ref, posa_ref, posb_ref, be_ref,
                *, ne, nblk):
    oa = oa_ref[...]
    ob = ob_ref[...]
    t = oa.shape[0]
    oab = oa.astype(jnp.bfloat16)
    obb = ob.astype(jnp.bfloat16)
    tio_r = jax.lax.broadcasted_iota(jnp.int32, (t, t), 0)
    tio_c = jax.lax.broadcasted_iota(jnp.int32, (t, t), 1)
    ls = (tio_c < tio_r).astype(jnp.bfloat16)
    ra = jnp.dot(ls, oab, preferred_element_type=jnp.float32)
    rb = jnp.dot(ls, obb, preferred_element_type=jnp.float32)
    tot_a = jnp.sum(oa, axis=0, keepdims=True)
    cnt = tot_a + jnp.sum(ob, axis=0, keepdims=True)
    blocks = jnp.floor((cnt + (BLKF - 1)) * (1.0 / BLKF))
    eiota_r = jax.lax.broadcasted_iota(jnp.int32, (ne, ne), 0)
    eiota_c = jax.lax.broadcasted_iota(jnp.int32, (ne, ne), 1)
    m8 = (eiota_r < eiota_c).astype(jnp.float32)
    sblk = jnp.dot(blocks, m8, preferred_element_type=jnp.float32)
    spad = sblk * BLKF
    posa = jnp.sum(oa * (ra + spad), axis=1)
    posb = jnp.sum(ob * (rb + tot_a + spad), axis=1)
    posa_ref[...] = posa.astype(jnp.int32)
    posb_ref[...] = posb.astype(jnp.int32)
    cnt_t = jax.lax.dot_general(oa + ob, jnp.ones((t, 1), jnp.float32),
                                (((0,), (0,)), ((), ())),
                                preferred_element_type=jnp.float32)
    blocks_t = jnp.floor((cnt_t + (BLKF - 1)) * (1.0 / BLKF))
    m8l = (eiota_c < eiota_r).astype(jnp.float32)
    sblk_t = jnp.dot(m8l, blocks_t, preferred_element_type=jnp.float32)
    biota = jax.lax.broadcasted_iota(jnp.int32, (ne, nblk), 1).astype(jnp.float32)
    ge = (biota >= sblk_t).astype(jnp.float32)
    be = jnp.sum(ge, axis=0, keepdims=True) - 1.0
    total = jnp.sum(blocks_t)
    biota1 = jax.lax.broadcasted_iota(jnp.int32, (1, nblk), 1).astype(jnp.float32)
    be = jnp.where(biota1 < total, be, -1.0)
    be_ref[...] = be[0].astype(jnp.int32)


def _gffn_body(be_ref, xg_ref, wg_ref, wi_ref, woe_ref, y_ref):
    b = pl.program_id(0)

    @pl.when(be_ref[b] >= 0)
    def _compute():
        mb = xg_ref[...].astype(jnp.bfloat16)
        g = jnp.dot(mb, wg_ref[0].astype(jnp.bfloat16),
                    preferred_element_type=jnp.float32)
        u = jnp.dot(mb, wi_ref[0].astype(jnp.bfloat16),
                    preferred_element_type=jnp.float32)
        h = (g * jax.nn.sigmoid(g) * u).astype(jnp.bfloat16)
        y_ref[...] = jnp.dot(h, woe_ref[0].astype(jnp.bfloat16),
                             preferred_element_type=jnp.float32)


def _sc_mesh():
    return plsc.VectorSubcoreMesh(core_axis_name="c", subcore_axis_name="s",
                                  num_cores=NC, num_subcores=NS)


def _sc_dispatch(pos_all, m, *, nrows, t, dim, nslots):
    nw = NC * NS
    spt = nslots // nw
    ch = spt // 2

    @functools.partial(
        pl.kernel,
        out_type=jax.ShapeDtypeStruct((nrows, dim), jnp.float32),
        mesh=_sc_mesh(),
        scratch_types=[pltpu.VMEM((ch,), jnp.int32),
                       pltpu.VMEM((ch,), jnp.int32),
                       pltpu.VMEM((ch, dim), jnp.float32),
                       pltpu.SemaphoreType.DMA],
        compiler_params=pltpu.CompilerParams(needs_layout_passes=False),
    )
    def _dispatch(pos_hbm, m_hbm, xg_hbm, idx_a, idx_b, rows_v, sem):
        wid = lax.axis_index("s") * NC + lax.axis_index("c")
        base = wid * spt
        tok0 = base % t
        pltpu.sync_copy(pos_hbm.at[pl.ds(base, ch)], idx_a)
        pltpu.sync_copy(pos_hbm.at[pl.ds(base + ch, ch)], idx_b)
        pltpu.sync_copy(m_hbm.at[pl.ds(tok0, ch)], rows_v)
        pltpu.async_copy(rows_v, xg_hbm.at[idx_a], sem).wait()
        pltpu.sync_copy(m_hbm.at[pl.ds(tok0 + ch, ch)], rows_v)
        pltpu.async_copy(rows_v, xg_hbm.at[idx_b], sem).wait()

    return _dispatch(pos_all, m)


def _sc_combine(pos_all, y, *, dim, nslots):
    nw = NC * NS
    spt = nslots // nw
    ch = spt // 2

    @functools.partial(
        pl.kernel,
        out_type=jax.ShapeDtypeStruct((nslots, dim), jnp.float32),
        mesh=_sc_mesh(),
        scratch_types=[pltpu.VMEM((ch,), jnp.int32),
                       pltpu.VMEM((ch,), jnp.int32),
                       pltpu.VMEM((ch, dim), jnp.float32),
                       pltpu.SemaphoreType.DMA],
        compiler_params=pltpu.CompilerParams(needs_layout_passes=False),
    )
    def _combine(pos_hbm, y_hbm, yg_hbm, idx_a, idx_b, rows_v, sem):
        wid = lax.axis_index("s") * NC + lax.axis_index("c")
        base = wid * spt
        pltpu.sync_copy(pos_hbm.at[pl.ds(base, ch)], idx_a)
        pltpu.sync_copy(pos_hbm.at[pl.ds(base + ch, ch)], idx_b)
        pltpu.async_copy(y_hbm.at[idx_a], rows_v, sem).wait()
        pltpu.sync_copy(rows_v, yg_hbm.at[pl.ds(base, ch)])
        pltpu.async_copy(y_hbm.at[idx_b], rows_v, sem).wait()
        pltpu.sync_copy(rows_v, yg_hbm.at[pl.ds(base + ch, ch)])

    return _combine(pos_all, y)


def _final_body(x2_ref, ya_ref, yb_ref, wa_ref, wb_ref, o_ref):
    wa = jnp.reshape(wa_ref[...], (-1, 1))
    wb = jnp.reshape(wb_ref[...], (-1, 1))
    o_ref[...] = x2_ref[...] + wa * ya_ref[...] + wb * yb_ref[...]


def kernel(x, attn_norm_w, Wq, Wk, Wv, Wo, q_norm_w, k_norm_w, ffn_norm_w,
           Wi, Wg, Woe, router_w, cos, sin):
    b, t, dim = x.shape
    nq = Wq.shape[1] // cos.shape[1]
    nkv = Wk.shape[1] // cos.shape[1]
    hd = cos.shape[1]
    ne, _, hid = Wi.shape
    blk = min(BLK, t)
    nt = t // blk
    nslots = 2 * t
    nblk = nslots // BLKF + ne
    nrows = nblk * BLKF

    x2d = x.reshape(t, dim)
    bf = jnp.bfloat16
    f32 = jnp.float32
    wi_b, wg_b, woe_b = Wi, Wg, Woe

    p64 = _rot_perm(hd)
    pq = jnp.asarray(np.kron(np.eye(nq, dtype=np.float32), p64), bf)
    pk = jnp.asarray(np.kron(np.eye(nkv, dtype=np.float32), p64), bf)
    hq = jnp.asarray(np.kron(np.eye(nq, dtype=np.float32),
                             np.ones((hd, 1), np.float32)))
    hk = jnp.asarray(np.kron(np.eye(nkv, dtype=np.float32),
                             np.ones((hd, 1), np.float32)))
    qnw = jnp.tile(q_norm_w, (nq,)).reshape(1, nq * hd)
    knw = jnp.tile(k_norm_w, (nkv,)).reshape(1, nkv * hd)
    anw = attn_norm_w.reshape(1, dim)
    fnw = ffn_norm_w.reshape(1, dim)

    dq, dkv = nq * hd, nkv * hd
    rep = nq // nkv

    full = lambda shape: pl.BlockSpec(shape, lambda *_: (0,) * len(shape))
    rowblk = lambda w: pl.BlockSpec((blk, w), lambda i: (i, 0))

    q, k, v = pl.pallas_call(
        functools.partial(_prelude_body, hd=hd, nq=nq, nkv=nkv),
        grid=(nt,),
        in_specs=[
            rowblk(dim), full((1, dim)), full((dim, dq)), full((dim, dkv)),
            full((dim, dkv)), full((1, dq)), full((1, dkv)),
            rowblk(hd), rowblk(hd),
            full((dq, dq)), full((dkv, dkv)),
            full((dq, nq)), full((nq, dq)), full((dkv, nkv)), full((nkv, dkv)),
        ],
        out_specs=[rowblk(dq), rowblk(dkv), rowblk(dkv)],
        out_shape=[
            jax.ShapeDtypeStruct((t, dq), bf),
            jax.ShapeDtypeStruct((t, dkv), bf),
            jax.ShapeDtypeStruct((t, dkv), bf),
        ],
    )(x2d, anw, Wq, Wk, Wv, qnw, knw, cos, sin,
      pq, pk, hq, hq.T, hk, hk.T)

    ctx = pl.pallas_call(
        functools.partial(_attn_body, hd=hd, nq=nq, rep=rep),
        grid=(nt,),
        in_specs=[rowblk(dq), full((t, dkv)), full((t, dkv))],
        out_specs=rowblk(dq),
        out_shape=jax.ShapeDtypeStruct((t, dq), bf),
    )(q, k, v)

    x2, m, oa, ob, wa, wb = pl.pallas_call(
        functools.partial(_post_body, ne=ne),
        grid=(nt,),
        in_specs=[rowblk(dq), full((dq, dim)), rowblk(dim), full((1, dim)),
                  full((dim, ne))],
        out_specs=[rowblk(dim), rowblk(dim), rowblk(ne), rowblk(ne),
                   pl.BlockSpec((blk,), lambda i: (i,)),
                   pl.BlockSpec((blk,), lambda i: (i,))],
        out_shape=[
            jax.ShapeDtypeStruct((t, dim), f32),
            jax.ShapeDtypeStruct((t, dim), f32),
            jax.ShapeDtypeStruct((t, ne), f32),
            jax.ShapeDtypeStruct((t, ne), f32),
            jax.ShapeDtypeStruct((t,), f32),
            jax.ShapeDtypeStruct((t,), f32),
        ],
    )(ctx, Wo, x2d, fnw, router_w)

    posa, posb, be = pl.pallas_call(
        functools.partial(_route_body, ne=ne, nblk=nblk),
        grid=(1,),
        in_specs=[full((t, ne)), full((t, ne))],
        out_specs=[pl.BlockSpec((t,), lambda i: (0,)),
                   pl.BlockSpec((t,), lambda i: (0,)),
                   pl.BlockSpec((nblk,), lambda i: (0,))],
        out_shape=[
            jax.ShapeDtypeStruct((t,), jnp.int32),
            jax.ShapeDtypeStruct((t,), jnp.int32),
            jax.ShapeDtypeStruct((nblk,), jnp.int32),
        ],
    )(oa, ob)

    pos_all = jnp.concatenate([posa, posb])
    xg = _sc_dispatch(pos_all, m, nrows=nrows, t=t, dim=dim, nslots=nslots)

    y = pl.pallas_call(
        _gffn_body,
        grid_spec=pltpu.PrefetchScalarGridSpec(
            num_scalar_prefetch=1,
            grid=(nblk,),
            in_specs=[
                pl.BlockSpec((BLKF, dim), lambda bi, be_s: (bi, 0)),
                pl.BlockSpec((1, dim, hid),
                             lambda bi, be_s: (jnp.maximum(be_s[bi], 0), 0, 0)),
                pl.BlockSpec((1, dim, hid),
                             lambda bi, be_s: (jnp.maximum(be_s[bi], 0), 0, 0)),
                pl.BlockSpec((1, hid, dim),
                             lambda bi, be_s: (jnp.maximum(be_s[bi], 0), 0, 0)),
            ],
            out_specs=pl.BlockSpec((BLKF, dim), lambda bi, be_s: (bi, 0)),
        ),
        out_shape=jax.ShapeDtypeStruct((nrows, dim), f32),
        compiler_params=pltpu.CompilerParams(
            dimension_semantics=("arbitrary",)),
    )(be, xg, wg_b, wi_b, woe_b)

    yg = _sc_combine(pos_all, y, dim=dim, nslots=nslots)
    ya, yb = yg[:t], yg[t:]

    out = pl.pallas_call(
        _final_body,
        grid=(nt,),
        in_specs=[rowblk(dim), rowblk(dim), rowblk(dim),
                  pl.BlockSpec((blk,), lambda i: (i,)),
                  pl.BlockSpec((blk,), lambda i: (i,))],
        out_specs=rowblk(dim),
        out_shape=jax.ShapeDtypeStruct((t, dim), f32),
    )(x2, ya, yb, wa, wb)

    return out.reshape(b, t, dim)


# speedup vs baseline: 2.8924x; 1.0983x over previous
import functools

import jax
import jax.numpy as jnp
import numpy as np
from jax import lax
from jax.experimental import pallas as pl
from jax.experimental.pallas import tpu as pltpu
from jax.experimental.pallas import tpu_sc as plsc

EPS = 1e-6
BLK = 256
BLKF = 128
NC, NS, L = 2, 16, 16


def _rot_perm(hd):
    h = hd // 2
    eye = np.eye(h, dtype=np.float32)
    z = np.zeros((h, h), np.float32)
    return np.block([[z, eye], [-eye, z]])


def _prelude_body(x_ref, anw_ref, wq_ref, wk_ref, wv_ref, qnw_ref, knw_ref,
                  c_ref, s_ref, pq_ref, pk_ref, hq_ref, hqt_ref,
                  hk_ref, hkt_ref, q_ref, k_ref, v_ref, *, hd, nq, nkv):
    xs = x_ref[...]
    a = xs * jax.lax.rsqrt(jnp.mean(xs * xs, axis=-1, keepdims=True) + EPS)
    a = (a * anw_ref[...]).astype(jnp.bfloat16)
    cb = c_ref[...]
    sb = s_ref[...]

    def qk_path(w_ref, nw_ref, p_ref, h_ref, ht_ref, nh):
        q = jnp.dot(a, w_ref[...].astype(jnp.bfloat16),
                    preferred_element_type=jnp.float32)
        ss = jnp.dot(q * q, h_ref[...], preferred_element_type=jnp.float32)
        rs = jax.lax.rsqrt(ss / hd + EPS)
        qn = q * jnp.dot(rs, ht_ref[...], preferred_element_type=jnp.float32)
        qn = qn * nw_ref[...]
        qr = jnp.dot(qn.astype(jnp.bfloat16), p_ref[...],
                     preferred_element_type=jnp.float32)
        c = jnp.concatenate([cb] * nh, axis=1)
        s = jnp.concatenate([sb] * nh, axis=1)
        return (qn * c + qr * s).astype(jnp.bfloat16)

    q_ref[...] = (qk_path(wq_ref, qnw_ref, pq_ref, hq_ref, hqt_ref, nq)
                  .astype(jnp.float32) * (1.0 / hd ** 0.5)).astype(jnp.bfloat16)
    k_ref[...] = qk_path(wk_ref, knw_ref, pk_ref, hk_ref, hkt_ref, nkv)
    v_ref[...] = jnp.dot(a, wv_ref[...].astype(jnp.bfloat16),
                         preferred_element_type=jnp.float32).astype(jnp.bfloat16)


def _attnpost_body(q_ref, k_ref, v_ref, wo_ref, x_ref, fnw_ref, rw_ref,
                   x2_ref, m_ref, oa_ref, ob_ref, wa_ref, wb_ref,
                   *, hd, nq, rep, ne):
    qs = q_ref[...]
    ks = k_ref[...]
    vs = v_ref[...]
    outs = []
    for h in range(nq):
        g = h // rep
        q = qs[:, h * hd:(h + 1) * hd]
        k = ks[:, g * hd:(g + 1) * hd]
        v = vs[:, g * hd:(g + 1) * hd]
        s = jax.lax.dot_general(q, k, (((1,), (1,)), ((), ())),
                                preferred_element_type=jnp.float32)
        m = jnp.max(s, axis=-1, keepdims=True)
        e = jnp.exp(s - m)
        r = 1.0 / jnp.sum(e, axis=-1, keepdims=True)
        c = jnp.dot(e.astype(jnp.bfloat16), v,
                    preferred_element_type=jnp.float32)
        outs.append((c * r).astype(jnp.bfloat16))
    ctx = jnp.concatenate(outs, axis=1)
    x2 = x_ref[...] + jnp.dot(ctx, wo_ref[...].astype(jnp.bfloat16),
                              preferred_element_type=jnp.float32)
    x2_ref[...] = x2
    mm = x2 * jax.lax.rsqrt(jnp.mean(x2 * x2, axis=-1, keepdims=True) + EPS)
    mm = mm * fnw_ref[...]
    m_ref[...] = mm
    logits = jnp.dot(mm, rw_ref[...], preferred_element_type=jnp.float32)
    mx = jnp.max(logits, axis=-1, keepdims=True)
    ex = jnp.exp(logits - mx)
    g = ex / jnp.sum(ex, axis=-1, keepdims=True)
    it = jax.lax.broadcasted_iota(jnp.int32, g.shape, 1)
    m1 = jnp.max(g, axis=-1, keepdims=True)
    i1 = jnp.min(jnp.where(g == m1, it, ne), axis=-1, keepdims=True)
    g2 = jnp.where(it == i1, -jnp.inf, g)
    m2 = jnp.max(g2, axis=-1, keepdims=True)
    i2 = jnp.min(jnp.where(g2 == m2, it, ne), axis=-1, keepdims=True)
    oa_ref[...] = (it == i1).astype(jnp.float32)
    ob_ref[...] = (it == i2).astype(jnp.float32)
    wa_ref[...] = m1[:, 0]
    wb_ref[...] = m2[:, 0]


def _route_body(oa_ref, ob_ref, posa_ref, posb_ref, be_ref,
                *, ne, nblk):
    oa = oa_ref[...]
    ob = ob_ref[...]
    t = oa.shape[0]
    oab = oa.astype(jnp.bfloat16)
    obb = ob.astype(jnp.bfloat16)
    tio_r = jax.lax.broadcasted_iota(jnp.int32, (t, t), 0)
    tio_c = jax.lax.broadcasted_iota(jnp.int32, (t, t), 1)
    ls = (tio_c < tio_r).astype(jnp.bfloat16)
    ra = jnp.dot(ls, oab, preferred_element_type=jnp.float32)
    rb = jnp.dot(ls, obb, preferred_element_type=jnp.float32)
    tot_a = jnp.sum(oa, axis=0, keepdims=True)
    cnt = tot_a + jnp.sum(ob, axis=0, keepdims=True)
    blocks = jnp.floor((cnt + (BLKF - 1)) * (1.0 / BLKF))
    eiota_r = jax.lax.broadcasted_iota(jnp.int32, (ne, ne), 0)
    eiota_c = jax.lax.broadcasted_iota(jnp.int32, (ne, ne), 1)
    m8 = (eiota_r < eiota_c).astype(jnp.float32)
    sblk = jnp.dot(blocks, m8, preferred_element_type=jnp.float32)
    spad = sblk * BLKF
    posa = jnp.sum(oa * (ra + spad), axis=1)
    posb = jnp.sum(ob * (rb + tot_a + spad), axis=1)
    posa_ref[...] = posa.astype(jnp.int32)
    posb_ref[...] = posb.astype(jnp.int32)
    cnt_t = jax.lax.dot_general(oa + ob, jnp.ones((t, 1), jnp.float32),
                                (((0,), (0,)), ((), ())),
                                preferred_element_type=jnp.float32)
    blocks_t = jnp.floor((cnt_t + (BLKF - 1)) * (1.0 / BLKF))
    m8l = (eiota_c < eiota_r).astype(jnp.float32)
    sblk_t = jnp.dot(m8l, blocks_t, preferred_element_type=jnp.float32)
    biota = jax.lax.broadcasted_iota(jnp.int32, (ne, nblk), 1).astype(jnp.float32)
    ge = (biota >= sblk_t).astype(jnp.float32)
    be = jnp.sum(ge, axis=0, keepdims=True) - 1.0
    total = jnp.sum(blocks_t)
    biota1 = jax.lax.broadcasted_iota(jnp.int32, (1, nblk), 1).astype(jnp.float32)
    be = jnp.where(biota1 < total, be, -1.0)
    be_ref[...] = be[0].astype(jnp.int32)


def _gffn_body(be_ref, xg_ref, wg_ref, wi_ref, woe_ref, y_ref):
    b = pl.program_id(0)

    @pl.when(be_ref[b] >= 0)
    def _compute():
        mb = xg_ref[...].astype(jnp.bfloat16)
        g = jnp.dot(mb, wg_ref[0].astype(jnp.bfloat16),
                    preferred_element_type=jnp.float32)
        u = jnp.dot(mb, wi_ref[0].astype(jnp.bfloat16),
                    preferred_element_type=jnp.float32)
        h = (g * jax.nn.sigmoid(g) * u).astype(jnp.bfloat16)
        y_ref[...] = jnp.dot(h, woe_ref[0].astype(jnp.bfloat16),
                             preferred_element_type=jnp.float32)


def _sc_mesh():
    return plsc.VectorSubcoreMesh(core_axis_name="c", subcore_axis_name="s",
                                  num_cores=NC, num_subcores=NS)


def _sc_dispatch(pos_all, m, *, nrows, t, dim, nslots):
    nw = NC * NS
    spt = nslots // nw
    ch = spt // 2

    @functools.partial(
        pl.kernel,
        out_type=jax.ShapeDtypeStruct((nrows, dim), jnp.float32),
        mesh=_sc_mesh(),
        scratch_types=[pltpu.VMEM((ch,), jnp.int32),
                       pltpu.VMEM((ch,), jnp.int32),
                       pltpu.VMEM((ch, dim), jnp.float32),
                       pltpu.SemaphoreType.DMA],
        compiler_params=pltpu.CompilerParams(needs_layout_passes=False),
    )
    def _dispatch(pos_hbm, m_hbm, xg_hbm, idx_a, idx_b, rows_v, sem):
        wid = lax.axis_index("s") * NC + lax.axis_index("c")
        base = wid * spt
        tok0 = base % t
        pltpu.sync_copy(pos_hbm.at[pl.ds(base, ch)], idx_a)
        pltpu.sync_copy(pos_hbm.at[pl.ds(base + ch, ch)], idx_b)
        pltpu.sync_copy(m_hbm.at[pl.ds(tok0, ch)], rows_v)
        pltpu.async_copy(rows_v, xg_hbm.at[idx_a], sem).wait()
        pltpu.sync_copy(m_hbm.at[pl.ds(tok0 + ch, ch)], rows_v)
        pltpu.async_copy(rows_v, xg_hbm.at[idx_b], sem).wait()

    return _dispatch(pos_all, m)


def _sc_combine(pos_all, y, *, dim, nslots):
    nw = NC * NS
    spt = nslots // nw
    ch = spt // 2

    @functools.partial(
        pl.kernel,
        out_type=jax.ShapeDtypeStruct((nslots, dim), jnp.float32),
        mesh=_sc_mesh(),
        scratch_types=[pltpu.VMEM((ch,), jnp.int32),
                       pltpu.VMEM((ch,), jnp.int32),
                       pltpu.VMEM((ch, dim), jnp.float32),
                       pltpu.SemaphoreType.DMA],
        compiler_params=pltpu.CompilerParams(needs_layout_passes=False),
    )
    def _combine(pos_hbm, y_hbm, yg_hbm, idx_a, idx_b, rows_v, sem):
        wid = lax.axis_index("s") * NC + lax.axis_index("c")
        base = wid * spt
        pltpu.sync_copy(pos_hbm.at[pl.ds(base, ch)], idx_a)
        pltpu.sync_copy(pos_hbm.at[pl.ds(base + ch, ch)], idx_b)
        pltpu.async_copy(y_hbm.at[idx_a], rows_v, sem).wait()
        pltpu.sync_copy(rows_v, yg_hbm.at[pl.ds(base, ch)])
        pltpu.async_copy(y_hbm.at[idx_b], rows_v, sem).wait()
        pltpu.sync_copy(rows_v, yg_hbm.at[pl.ds(base + ch, ch)])

    return _combine(pos_all, y)


def _final_body(x2_ref, ya_ref, yb_ref, wa_ref, wb_ref, o_ref):
    wa = jnp.reshape(wa_ref[...], (-1, 1))
    wb = jnp.reshape(wb_ref[...], (-1, 1))
    o_ref[...] = x2_ref[...] + wa * ya_ref[...] + wb * yb_ref[...]


def kernel(x, attn_norm_w, Wq, Wk, Wv, Wo, q_norm_w, k_norm_w, ffn_norm_w,
           Wi, Wg, Woe, router_w, cos, sin):
    b, t, dim = x.shape
    nq = Wq.shape[1] // cos.shape[1]
    nkv = Wk.shape[1] // cos.shape[1]
    hd = cos.shape[1]
    ne, _, hid = Wi.shape
    blk = min(BLK, t)
    nt = t // blk
    nslots = 2 * t
    nblk = nslots // BLKF + ne
    nrows = nblk * BLKF

    x2d = x.reshape(t, dim)
    bf = jnp.bfloat16
    f32 = jnp.float32
    wi_b, wg_b, woe_b = Wi, Wg, Woe

    p64 = _rot_perm(hd)
    pq = jnp.asarray(np.kron(np.eye(nq, dtype=np.float32), p64), bf)
    pk = jnp.asarray(np.kron(np.eye(nkv, dtype=np.float32), p64), bf)
    hq = jnp.asarray(np.kron(np.eye(nq, dtype=np.float32),
                             np.ones((hd, 1), np.float32)))
    hk = jnp.asarray(np.kron(np.eye(nkv, dtype=np.float32),
                             np.ones((hd, 1), np.float32)))
    qnw = jnp.tile(q_norm_w, (nq,)).reshape(1, nq * hd)
    knw = jnp.tile(k_norm_w, (nkv,)).reshape(1, nkv * hd)
    anw = attn_norm_w.reshape(1, dim)
    fnw = ffn_norm_w.reshape(1, dim)

    dq, dkv = nq * hd, nkv * hd
    rep = nq // nkv

    full = lambda shape: pl.BlockSpec(shape, lambda *_: (0,) * len(shape))
    rowblk = lambda w: pl.BlockSpec((blk, w), lambda i: (i, 0))

    q, k, v = pl.pallas_call(
        functools.partial(_prelude_body, hd=hd, nq=nq, nkv=nkv),
        grid=(nt,),
        in_specs=[
            rowblk(dim), full((1, dim)), full((dim, dq)), full((dim, dkv)),
            full((dim, dkv)), full((1, dq)), full((1, dkv)),
            rowblk(hd), rowblk(hd),
            full((dq, dq)), full((dkv, dkv)),
            full((dq, nq)), full((nq, dq)), full((dkv, nkv)), full((nkv, dkv)),
        ],
        out_specs=[rowblk(dq), rowblk(dkv), rowblk(dkv)],
        out_shape=[
            jax.ShapeDtypeStruct((t, dq), bf),
            jax.ShapeDtypeStruct((t, dkv), bf),
            jax.ShapeDtypeStruct((t, dkv), bf),
        ],
    )(x2d, anw, Wq, Wk, Wv, qnw, knw, cos, sin,
      pq, pk, hq, hq.T, hk, hk.T)

    x2, m, oa, ob, wa, wb = pl.pallas_call(
        functools.partial(_attnpost_body, hd=hd, nq=nq, rep=rep, ne=ne),
        grid=(nt,),
        in_specs=[rowblk(dq), full((t, dkv)), full((t, dkv)),
                  full((dq, dim)), rowblk(dim), full((1, dim)),
                  full((dim, ne))],
        out_specs=[rowblk(dim), rowblk(dim), rowblk(ne), rowblk(ne),
                   pl.BlockSpec((blk,), lambda i: (i,)),
                   pl.BlockSpec((blk,), lambda i: (i,))],
        out_shape=[
            jax.ShapeDtypeStruct((t, dim), f32),
            jax.ShapeDtypeStruct((t, dim), f32),
            jax.ShapeDtypeStruct((t, ne), f32),
            jax.ShapeDtypeStruct((t, ne), f32),
            jax.ShapeDtypeStruct((t,), f32),
            jax.ShapeDtypeStruct((t,), f32),
        ],
    )(q, k, v, Wo, x2d, fnw, router_w)

    posa, posb, be = pl.pallas_call(
        functools.partial(_route_body, ne=ne, nblk=nblk),
        grid=(1,),
        in_specs=[full((t, ne)), full((t, ne))],
        out_specs=[pl.BlockSpec((t,), lambda i: (0,)),
                   pl.BlockSpec((t,), lambda i: (0,)),
                   pl.BlockSpec((nblk,), lambda i: (0,))],
        out_shape=[
            jax.ShapeDtypeStruct((t,), jnp.int32),
            jax.ShapeDtypeStruct((t,), jnp.int32),
            jax.ShapeDtypeStruct((nblk,), jnp.int32),
        ],
    )(oa, ob)

    pos_all = jnp.concatenate([posa, posb])
    xg = _sc_dispatch(pos_all, m, nrows=nrows, t=t, dim=dim, nslots=nslots)

    y = pl.pallas_call(
        _gffn_body,
        grid_spec=pltpu.PrefetchScalarGridSpec(
            num_scalar_prefetch=1,
            grid=(nblk,),
            in_specs=[
                pl.BlockSpec((BLKF, dim), lambda bi, be_s: (bi, 0)),
                pl.BlockSpec((1, dim, hid),
                             lambda bi, be_s: (jnp.maximum(be_s[bi], 0), 0, 0)),
                pl.BlockSpec((1, dim, hid),
                             lambda bi, be_s: (jnp.maximum(be_s[bi], 0), 0, 0)),
                pl.BlockSpec((1, hid, dim),
                             lambda bi, be_s: (jnp.maximum(be_s[bi], 0), 0, 0)),
            ],
            out_specs=pl.BlockSpec((BLKF, dim), lambda bi, be_s: (bi, 0)),
        ),
        out_shape=jax.ShapeDtypeStruct((nrows, dim), f32),
        compiler_params=pltpu.CompilerParams(
            dimension_semantics=("arbitrary",)),
    )(be, xg, wg_b, wi_b, woe_b)

    yg = _sc_combine(pos_all, y, dim=dim, nslots=nslots)
    ya, yb = yg[:t], yg[t:]

    out = pl.pallas_call(
        _final_body,
        grid=(nt,),
        in_specs=[rowblk(dim), rowblk(dim), rowblk(dim),
                  pl.BlockSpec((blk,), lambda i: (i,)),
                  pl.BlockSpec((blk,), lambda i: (i,))],
        out_specs=rowblk(dim),
        out_shape=jax.ShapeDtypeStruct((t, dim), f32),
    )(x2, ya, yb, wa, wb)

    return out.reshape(b, t, dim)


# speedup vs baseline: 2.9031x; 1.0037x over previous
import functools

import jax
import jax.numpy as jnp
import numpy as np
from jax import lax
from jax.experimental import pallas as pl
from jax.experimental.pallas import tpu as pltpu
from jax.experimental.pallas import tpu_sc as plsc

EPS = 1e-6
BLK = 256
BLKF = 128
NC, NS, L = 2, 16, 16


def _rot_perm(hd):
    h = hd // 2
    eye = np.eye(h, dtype=np.float32)
    z = np.zeros((h, h), np.float32)
    return np.block([[z, eye], [-eye, z]])


def _prelude_body(x_ref, anw_ref, wq_ref, wk_ref, wv_ref, qnw_ref, knw_ref,
                  c_ref, s_ref, pq_ref, pk_ref, hq_ref, hqt_ref,
                  hk_ref, hkt_ref, q_ref, k_ref, v_ref, *, hd, nq, nkv):
    xs = x_ref[...]
    a = xs * jax.lax.rsqrt(jnp.mean(xs * xs, axis=-1, keepdims=True) + EPS)
    a = (a * anw_ref[...]).astype(jnp.bfloat16)
    cb = c_ref[...]
    sb = s_ref[...]

    def qk_path(w_ref, nw_ref, p_ref, h_ref, ht_ref, nh):
        q = jnp.dot(a, w_ref[...].astype(jnp.bfloat16),
                    preferred_element_type=jnp.float32)
        ss = jnp.dot(q * q, h_ref[...], preferred_element_type=jnp.float32)
        rs = jax.lax.rsqrt(ss / hd + EPS)
        qn = q * jnp.dot(rs, ht_ref[...], preferred_element_type=jnp.float32)
        qn = qn * nw_ref[...]
        qr = jnp.dot(qn.astype(jnp.bfloat16), p_ref[...],
                     preferred_element_type=jnp.float32)
        c = jnp.concatenate([cb] * nh, axis=1)
        s = jnp.concatenate([sb] * nh, axis=1)
        return (qn * c + qr * s).astype(jnp.bfloat16)

    q_ref[...] = (qk_path(wq_ref, qnw_ref, pq_ref, hq_ref, hqt_ref, nq)
                  .astype(jnp.float32) * (1.0 / hd ** 0.5)).astype(jnp.bfloat16)
    k_ref[...] = qk_path(wk_ref, knw_ref, pk_ref, hk_ref, hkt_ref, nkv)
    v_ref[...] = jnp.dot(a, wv_ref[...].astype(jnp.bfloat16),
                         preferred_element_type=jnp.float32).astype(jnp.bfloat16)


def _attnpost_body(q_ref, k_ref, v_ref, wo_ref, x_ref, fnw_ref, rw_ref,
                   x2_ref, m_ref, wa_ref, wb_ref, posa_ref, posb_ref, be_ref,
                   oa_ref, ob_ref, *, hd, nq, rep, ne, nblk, nt, blk):
    qs = q_ref[...]
    ks = k_ref[...]
    vs = v_ref[...]
    outs = []
    for h in range(nq):
        g = h // rep
        q = qs[:, h * hd:(h + 1) * hd]
        k = ks[:, g * hd:(g + 1) * hd]
        v = vs[:, g * hd:(g + 1) * hd]
        s = jax.lax.dot_general(q, k, (((1,), (1,)), ((), ())),
                                preferred_element_type=jnp.float32)
        m = jnp.max(s, axis=-1, keepdims=True)
        e = jnp.exp(s - m)
        r = 1.0 / jnp.sum(e, axis=-1, keepdims=True)
        c = jnp.dot(e.astype(jnp.bfloat16), v,
                    preferred_element_type=jnp.float32)
        outs.append((c * r).astype(jnp.bfloat16))
    ctx = jnp.concatenate(outs, axis=1)
    x2 = x_ref[...] + jnp.dot(ctx, wo_ref[...].astype(jnp.bfloat16),
                              preferred_element_type=jnp.float32)
    x2_ref[...] = x2
    mm = x2 * jax.lax.rsqrt(jnp.mean(x2 * x2, axis=-1, keepdims=True) + EPS)
    mm = mm * fnw_ref[...]
    m_ref[...] = mm
    logits = jnp.dot(mm, rw_ref[...], preferred_element_type=jnp.float32)
    mx = jnp.max(logits, axis=-1, keepdims=True)
    ex = jnp.exp(logits - mx)
    g = ex / jnp.sum(ex, axis=-1, keepdims=True)
    it = jax.lax.broadcasted_iota(jnp.int32, g.shape, 1)
    m1 = jnp.max(g, axis=-1, keepdims=True)
    i1 = jnp.min(jnp.where(g == m1, it, ne), axis=-1, keepdims=True)
    g2 = jnp.where(it == i1, -jnp.inf, g)
    m2 = jnp.max(g2, axis=-1, keepdims=True)
    i2 = jnp.min(jnp.where(g2 == m2, it, ne), axis=-1, keepdims=True)
    i = pl.program_id(0)
    oa_ref[pl.ds(i * blk, blk), :] = (it == i1).astype(jnp.float32)
    ob_ref[pl.ds(i * blk, blk), :] = (it == i2).astype(jnp.float32)
    wa_ref[...] = m1[:, 0]
    wb_ref[...] = m2[:, 0]

    @pl.when(i == nt - 1)
    def _route():
        _route_math(oa_ref[...], ob_ref[...], posa_ref, posb_ref, be_ref,
                    ne=ne, nblk=nblk)


def _route_math(oa, ob, posa_ref, posb_ref, be_ref, *, ne, nblk):
    t = oa.shape[0]
    oab = oa.astype(jnp.bfloat16)
    obb = ob.astype(jnp.bfloat16)
    tio_r = jax.lax.broadcasted_iota(jnp.int32, (t, t), 0)
    tio_c = jax.lax.broadcasted_iota(jnp.int32, (t, t), 1)
    ls = (tio_c < tio_r).astype(jnp.bfloat16)
    ra = jnp.dot(ls, oab, preferred_element_type=jnp.float32)
    rb = jnp.dot(ls, obb, preferred_element_type=jnp.float32)
    tot_a = jnp.sum(oa, axis=0, keepdims=True)
    cnt = tot_a + jnp.sum(ob, axis=0, keepdims=True)
    blocks = jnp.floor((cnt + (BLKF - 1)) * (1.0 / BLKF))
    eiota_r = jax.lax.broadcasted_iota(jnp.int32, (ne, ne), 0)
    eiota_c = jax.lax.broadcasted_iota(jnp.int32, (ne, ne), 1)
    m8 = (eiota_r < eiota_c).astype(jnp.float32)
    sblk = jnp.dot(blocks, m8, preferred_element_type=jnp.float32)
    spad = sblk * BLKF
    posa = jnp.sum(oa * (ra + spad), axis=1)
    posb = jnp.sum(ob * (rb + tot_a + spad), axis=1)
    posa_ref[...] = posa.astype(jnp.int32)
    posb_ref[...] = posb.astype(jnp.int32)
    cnt_t = jax.lax.dot_general(oa + ob, jnp.ones((t, 1), jnp.float32),
                                (((0,), (0,)), ((), ())),
                                preferred_element_type=jnp.float32)
    blocks_t = jnp.floor((cnt_t + (BLKF - 1)) * (1.0 / BLKF))
    m8l = (eiota_c < eiota_r).astype(jnp.float32)
    sblk_t = jnp.dot(m8l, blocks_t, preferred_element_type=jnp.float32)
    biota = jax.lax.broadcasted_iota(jnp.int32, (ne, nblk), 1).astype(jnp.float32)
    ge = (biota >= sblk_t).astype(jnp.float32)
    be = jnp.sum(ge, axis=0, keepdims=True) - 1.0
    total = jnp.sum(blocks_t)
    biota1 = jax.lax.broadcasted_iota(jnp.int32, (1, nblk), 1).astype(jnp.float32)
    be = jnp.where(biota1 < total, be, -1.0)
    be_ref[...] = be[0].astype(jnp.int32)


def _gffn_body(be_ref, xg_ref, wg_ref, wi_ref, woe_ref, y_ref):
    b = pl.program_id(0)

    @pl.when(be_ref[b] >= 0)
    def _compute():
        mb = xg_ref[...].astype(jnp.bfloat16)
        g = jnp.dot(mb, wg_ref[0].astype(jnp.bfloat16),
                    preferred_element_type=jnp.float32)
        u = jnp.dot(mb, wi_ref[0].astype(jnp.bfloat16),
                    preferred_element_type=jnp.float32)
        h = (g * jax.nn.sigmoid(g) * u).astype(jnp.bfloat16)
        y_ref[...] = jnp.dot(h, woe_ref[0].astype(jnp.bfloat16),
                             preferred_element_type=jnp.float32)


def _sc_mesh():
    return plsc.VectorSubcoreMesh(core_axis_name="c", subcore_axis_name="s",
                                  num_cores=NC, num_subcores=NS)


def _sc_dispatch(pos_all, m, *, nrows, t, dim, nslots):
    nw = NC * NS
    spt = nslots // nw
    ch = spt // 2

    @functools.partial(
        pl.kernel,
        out_type=jax.ShapeDtypeStruct((nrows, dim), jnp.float32),
        mesh=_sc_mesh(),
        scratch_types=[pltpu.VMEM((ch,), jnp.int32),
                       pltpu.VMEM((ch,), jnp.int32),
                       pltpu.VMEM((ch, dim), jnp.float32),
                       pltpu.SemaphoreType.DMA],
        compiler_params=pltpu.CompilerParams(needs_layout_passes=False),
    )
    def _dispatch(pos_hbm, m_hbm, xg_hbm, idx_a, idx_b, rows_v, sem):
        wid = lax.axis_index("s") * NC + lax.axis_index("c")
        base = wid * spt
        tok0 = base % t
        pltpu.sync_copy(pos_hbm.at[pl.ds(base, ch)], idx_a)
        pltpu.sync_copy(pos_hbm.at[pl.ds(base + ch, ch)], idx_b)
        pltpu.sync_copy(m_hbm.at[pl.ds(tok0, ch)], rows_v)
        pltpu.async_copy(rows_v, xg_hbm.at[idx_a], sem).wait()
        pltpu.sync_copy(m_hbm.at[pl.ds(tok0 + ch, ch)], rows_v)
        pltpu.async_copy(rows_v, xg_hbm.at[idx_b], sem).wait()

    return _dispatch(pos_all, m)


def _sc_combine(pos_all, y, *, dim, nslots):
    nw = NC * NS
    spt = nslots // nw
    ch = spt // 2

    @functools.partial(
        pl.kernel,
        out_type=jax.ShapeDtypeStruct((nslots, dim), jnp.float32),
        mesh=_sc_mesh(),
        scratch_types=[pltpu.VMEM((ch,), jnp.int32),
                       pltpu.VMEM((ch,), jnp.int32),
                       pltpu.VMEM((ch, dim), jnp.float32),
                       pltpu.SemaphoreType.DMA],
        compiler_params=pltpu.CompilerParams(needs_layout_passes=False),
    )
    def _combine(pos_hbm, y_hbm, yg_hbm, idx_a, idx_b, rows_v, sem):
        wid = lax.axis_index("s") * NC + lax.axis_index("c")
        base = wid * spt
        pltpu.sync_copy(pos_hbm.at[pl.ds(base, ch)], idx_a)
        pltpu.sync_copy(pos_hbm.at[pl.ds(base + ch, ch)], idx_b)
        pltpu.async_copy(y_hbm.at[idx_a], rows_v, sem).wait()
        pltpu.sync_copy(rows_v, yg_hbm.at[pl.ds(base, ch)])
        pltpu.async_copy(y_hbm.at[idx_b], rows_v, sem).wait()
        pltpu.sync_copy(rows_v, yg_hbm.at[pl.ds(base + ch, ch)])

    return _combine(pos_all, y)


def _final_body(x2_ref, ya_ref, yb_ref, wa_ref, wb_ref, o_ref):
    wa = jnp.reshape(wa_ref[...], (-1, 1))
    wb = jnp.reshape(wb_ref[...], (-1, 1))
    o_ref[...] = x2_ref[...] + wa * ya_ref[...] + wb * yb_ref[...]


def kernel(x, attn_norm_w, Wq, Wk, Wv, Wo, q_norm_w, k_norm_w, ffn_norm_w,
           Wi, Wg, Woe, router_w, cos, sin):
    b, t, dim = x.shape
    nq = Wq.shape[1] // cos.shape[1]
    nkv = Wk.shape[1] // cos.shape[1]
    hd = cos.shape[1]
    ne, _, hid = Wi.shape
    blk = min(BLK, t)
    nt = t // blk
    nslots = 2 * t
    nblk = nslots // BLKF + ne
    nrows = nblk * BLKF

    x2d = x.reshape(t, dim)
    bf = jnp.bfloat16
    f32 = jnp.float32
    wi_b, wg_b, woe_b = Wi, Wg, Woe

    p64 = _rot_perm(hd)
    pq = jnp.asarray(np.kron(np.eye(nq, dtype=np.float32), p64), bf)
    pk = jnp.asarray(np.kron(np.eye(nkv, dtype=np.float32), p64), bf)
    hq = jnp.asarray(np.kron(np.eye(nq, dtype=np.float32),
                             np.ones((hd, 1), np.float32)))
    hk = jnp.asarray(np.kron(np.eye(nkv, dtype=np.float32),
                             np.ones((hd, 1), np.float32)))
    qnw = jnp.tile(q_norm_w, (nq,)).reshape(1, nq * hd)
    knw = jnp.tile(k_norm_w, (nkv,)).reshape(1, nkv * hd)
    anw = attn_norm_w.reshape(1, dim)
    fnw = ffn_norm_w.reshape(1, dim)

    dq, dkv = nq * hd, nkv * hd
    rep = nq // nkv

    full = lambda shape: pl.BlockSpec(shape, lambda *_: (0,) * len(shape))
    rowblk = lambda w: pl.BlockSpec((blk, w), lambda i: (i, 0))

    q, k, v = pl.pallas_call(
        functools.partial(_prelude_body, hd=hd, nq=nq, nkv=nkv),
        grid=(nt,),
        in_specs=[
            rowblk(dim), full((1, dim)), full((dim, dq)), full((dim, dkv)),
            full((dim, dkv)), full((1, dq)), full((1, dkv)),
            rowblk(hd), rowblk(hd),
            full((dq, dq)), full((dkv, dkv)),
            full((dq, nq)), full((nq, dq)), full((dkv, nkv)), full((nkv, dkv)),
        ],
        out_specs=[rowblk(dq), rowblk(dkv), rowblk(dkv)],
        out_shape=[
            jax.ShapeDtypeStruct((t, dq), bf),
            jax.ShapeDtypeStruct((t, dkv), bf),
            jax.ShapeDtypeStruct((t, dkv), bf),
        ],
    )(x2d, anw, Wq, Wk, Wv, qnw, knw, cos, sin,
      pq, pk, hq, hq.T, hk, hk.T)

    x2, m, wa, wb, posa, posb, be = pl.pallas_call(
        functools.partial(_attnpost_body, hd=hd, nq=nq, rep=rep, ne=ne,
                          nblk=nblk, nt=nt, blk=blk),
        grid=(nt,),
        in_specs=[rowblk(dq), full((t, dkv)), full((t, dkv)),
                  full((dq, dim)), rowblk(dim), full((1, dim)),
                  full((dim, ne))],
        out_specs=[rowblk(dim), rowblk(dim),
                   pl.BlockSpec((blk,), lambda i: (i,)),
                   pl.BlockSpec((blk,), lambda i: (i,)),
                   pl.BlockSpec((t,), lambda i: (0,)),
                   pl.BlockSpec((t,), lambda i: (0,)),
                   pl.BlockSpec((nblk,), lambda i: (0,))],
        out_shape=[
            jax.ShapeDtypeStruct((t, dim), f32),
            jax.ShapeDtypeStruct((t, dim), f32),
            jax.ShapeDtypeStruct((t,), f32),
            jax.ShapeDtypeStruct((t,), f32),
            jax.ShapeDtypeStruct((t,), jnp.int32),
            jax.ShapeDtypeStruct((t,), jnp.int32),
            jax.ShapeDtypeStruct((nblk,), jnp.int32),
        ],
        scratch_shapes=[pltpu.VMEM((t, ne), f32), pltpu.VMEM((t, ne), f32)],
    )(q, k, v, Wo, x2d, fnw, router_w)

    pos_all = jnp.concatenate([posa, posb])
    xg = _sc_dispatch(pos_all, m, nrows=nrows, t=t, dim=dim, nslots=nslots)

    y = pl.pallas_call(
        _gffn_body,
        grid_spec=pltpu.PrefetchScalarGridSpec(
            num_scalar_prefetch=1,
            grid=(nblk,),
            in_specs=[
                pl.BlockSpec((BLKF, dim), lambda bi, be_s: (bi, 0)),
                pl.BlockSpec((1, dim, hid),
                             lambda bi, be_s: (jnp.maximum(be_s[bi], 0), 0, 0)),
                pl.BlockSpec((1, dim, hid),
                             lambda bi, be_s: (jnp.maximum(be_s[bi], 0), 0, 0)),
                pl.BlockSpec((1, hid, dim),
                             lambda bi, be_s: (jnp.maximum(be_s[bi], 0), 0, 0)),
            ],
            out_specs=pl.BlockSpec((BLKF, dim), lambda bi, be_s: (bi, 0)),
        ),
        out_shape=jax.ShapeDtypeStruct((nrows, dim), f32),
        compiler_params=pltpu.CompilerParams(
            dimension_semantics=("arbitrary",)),
    )(be, xg, wg_b, wi_b, woe_b)

    yg = _sc_combine(pos_all, y, dim=dim, nslots=nslots)
    ya, yb = yg[:t], yg[t:]

    out = pl.pallas_call(
        _final_body,
        grid=(nt,),
        in_specs=[rowblk(dim), rowblk(dim), rowblk(dim),
                  pl.BlockSpec((blk,), lambda i: (i,)),
                  pl.BlockSpec((blk,), lambda i: (i,))],
        out_specs=rowblk(dim),
        out_shape=jax.ShapeDtypeStruct((t, dim), f32),
    )(x2, ya, yb, wa, wb)

    return out.reshape(b, t, dim)


# speedup vs baseline: 3.2341x; 1.1140x over previous
import functools

import jax
import jax.numpy as jnp
import numpy as np
from jax import lax
from jax.experimental import pallas as pl
from jax.experimental.pallas import tpu as pltpu
from jax.experimental.pallas import tpu_sc as plsc

EPS = 1e-6
BLK = 256
BLKF = 128
NC, NS, L = 2, 16, 16


def _rot_perm(hd):
    h = hd // 2
    eye = np.eye(h, dtype=np.float32)
    z = np.zeros((h, h), np.float32)
    return np.block([[z, eye], [-eye, z]])


def _prelude_body(x_ref, anw_ref, wq_ref, wk_ref, wv_ref, qnw_ref, knw_ref,
                  c_ref, s_ref, pq_ref, pk_ref, hq_ref, hqt_ref,
                  hk_ref, hkt_ref, q_ref, k_ref, v_ref, *, hd, nq, nkv):
    xs = x_ref[...]
    a = xs * jax.lax.rsqrt(jnp.mean(xs * xs, axis=-1, keepdims=True) + EPS)
    a = (a * anw_ref[...]).astype(jnp.bfloat16)
    cb = c_ref[...]
    sb = s_ref[...]

    def qk_path(w_ref, nw_ref, p_ref, h_ref, ht_ref, nh):
        q = jnp.dot(a, w_ref[...].astype(jnp.bfloat16),
                    preferred_element_type=jnp.float32)
        ss = jnp.dot(q * q, h_ref[...], preferred_element_type=jnp.float32)
        rs = jax.lax.rsqrt(ss / hd + EPS)
        qn = q * jnp.dot(rs, ht_ref[...], preferred_element_type=jnp.float32)
        qn = qn * nw_ref[...]
        qr = jnp.dot(qn.astype(jnp.bfloat16), p_ref[...],
                     preferred_element_type=jnp.float32)
        c = jnp.concatenate([cb] * nh, axis=1)
        s = jnp.concatenate([sb] * nh, axis=1)
        return (qn * c + qr * s).astype(jnp.bfloat16)

    q_ref[...] = (qk_path(wq_ref, qnw_ref, pq_ref, hq_ref, hqt_ref, nq)
                  .astype(jnp.float32) * (1.0 / hd ** 0.5)).astype(jnp.bfloat16)
    k_ref[...] = qk_path(wk_ref, knw_ref, pk_ref, hk_ref, hkt_ref, nkv)
    v_ref[...] = jnp.dot(a, wv_ref[...].astype(jnp.bfloat16),
                         preferred_element_type=jnp.float32).astype(jnp.bfloat16)


def _attnpost_body(q_ref, k_ref, v_ref, wo_ref, x_ref, fnw_ref, rw_ref,
                   x2_ref, m_ref, wa_ref, wb_ref, posa_ref, posb_ref, be_ref,
                   oa_ref, ob_ref, *, hd, nq, rep, ne, nblk, nt, blk):
    qs = q_ref[...]
    ks = k_ref[...]
    vs = v_ref[...]
    outs = []
    for h in range(nq):
        g = h // rep
        q = qs[:, h * hd:(h + 1) * hd]
        k = ks[:, g * hd:(g + 1) * hd]
        v = vs[:, g * hd:(g + 1) * hd]
        s = jax.lax.dot_general(q, k, (((1,), (1,)), ((), ())),
                                preferred_element_type=jnp.float32)
        e = jnp.exp(s)
        r = 1.0 / jnp.sum(e, axis=-1, keepdims=True)
        c = jnp.dot(e.astype(jnp.bfloat16), v,
                    preferred_element_type=jnp.float32)
        outs.append((c * r).astype(jnp.bfloat16))
    ctx = jnp.concatenate(outs, axis=1)
    x2 = x_ref[...] + jnp.dot(ctx, wo_ref[...].astype(jnp.bfloat16),
                              preferred_element_type=jnp.float32)
    x2_ref[...] = x2
    mm = x2 * jax.lax.rsqrt(jnp.mean(x2 * x2, axis=-1, keepdims=True) + EPS)
    mm = mm * fnw_ref[...]
    m_ref[...] = mm
    logits = jnp.dot(mm, rw_ref[...], preferred_element_type=jnp.float32)
    mx = jnp.max(logits, axis=-1, keepdims=True)
    ex = jnp.exp(logits - mx)
    g = ex / jnp.sum(ex, axis=-1, keepdims=True)
    it = jax.lax.broadcasted_iota(jnp.int32, g.shape, 1)
    m1 = jnp.max(g, axis=-1, keepdims=True)
    i1 = jnp.min(jnp.where(g == m1, it, ne), axis=-1, keepdims=True)
    g2 = jnp.where(it == i1, -jnp.inf, g)
    m2 = jnp.max(g2, axis=-1, keepdims=True)
    i2 = jnp.min(jnp.where(g2 == m2, it, ne), axis=-1, keepdims=True)
    i = pl.program_id(0)
    oa_ref[pl.ds(i * blk, blk), :] = (it == i1).astype(jnp.float32)
    ob_ref[pl.ds(i * blk, blk), :] = (it == i2).astype(jnp.float32)
    wa_ref[...] = m1[:, 0]
    wb_ref[...] = m2[:, 0]

    @pl.when(i == nt - 1)
    def _route():
        _route_math(oa_ref[...], ob_ref[...], posa_ref, posb_ref, be_ref,
                    ne=ne, nblk=nblk)


def _route_math(oa, ob, posa_ref, posb_ref, be_ref, *, ne, nblk):
    t = oa.shape[0]
    oab = oa.astype(jnp.bfloat16)
    obb = ob.astype(jnp.bfloat16)
    tio_r = jax.lax.broadcasted_iota(jnp.int32, (t, t), 0)
    tio_c = jax.lax.broadcasted_iota(jnp.int32, (t, t), 1)
    ls = (tio_c < tio_r).astype(jnp.bfloat16)
    ra = jnp.dot(ls, oab, preferred_element_type=jnp.float32)
    rb = jnp.dot(ls, obb, preferred_element_type=jnp.float32)
    tot_a = jnp.sum(oa, axis=0, keepdims=True)
    cnt = tot_a + jnp.sum(ob, axis=0, keepdims=True)
    blocks = jnp.floor((cnt + (BLKF - 1)) * (1.0 / BLKF))
    eiota_r = jax.lax.broadcasted_iota(jnp.int32, (ne, ne), 0)
    eiota_c = jax.lax.broadcasted_iota(jnp.int32, (ne, ne), 1)
    m8 = (eiota_r < eiota_c).astype(jnp.float32)
    sblk = jnp.dot(blocks, m8, preferred_element_type=jnp.float32)
    spad = sblk * BLKF
    posa = jnp.sum(oa * (ra + spad), axis=1)
    posb = jnp.sum(ob * (rb + tot_a + spad), axis=1)
    posa_ref[...] = posa.astype(jnp.int32)
    posb_ref[...] = posb.astype(jnp.int32)
    cnt_t = jax.lax.dot_general(oa + ob, jnp.ones((t, 1), jnp.float32),
                                (((0,), (0,)), ((), ())),
                                preferred_element_type=jnp.float32)
    blocks_t = jnp.floor((cnt_t + (BLKF - 1)) * (1.0 / BLKF))
    m8l = (eiota_c < eiota_r).astype(jnp.float32)
    sblk_t = jnp.dot(m8l, blocks_t, preferred_element_type=jnp.float32)
    biota = jax.lax.broadcasted_iota(jnp.int32, (ne, nblk), 1).astype(jnp.float32)
    ge = (biota >= sblk_t).astype(jnp.float32)
    be = jnp.sum(ge, axis=0, keepdims=True) - 1.0
    total = jnp.sum(blocks_t)
    biota1 = jax.lax.broadcasted_iota(jnp.int32, (1, nblk), 1).astype(jnp.float32)
    be = jnp.where(biota1 < total, be, -1.0)
    be_ref[...] = be[0].astype(jnp.int32)


def _gffn_body(be_ref, xg_ref, wg_ref, wi_ref, woe_ref, y_ref):
    b = pl.program_id(0)

    @pl.when(be_ref[b] >= 0)
    def _compute():
        mb = xg_ref[...].astype(jnp.bfloat16)
        g = jnp.dot(mb, wg_ref[0].astype(jnp.bfloat16),
                    preferred_element_type=jnp.float32)
        u = jnp.dot(mb, wi_ref[0].astype(jnp.bfloat16),
                    preferred_element_type=jnp.float32)
        h = (g * jax.nn.sigmoid(g) * u).astype(jnp.bfloat16)
        y_ref[...] = jnp.dot(h, woe_ref[0].astype(jnp.bfloat16),
                             preferred_element_type=jnp.float32)


def _sc_mesh():
    return plsc.VectorSubcoreMesh(core_axis_name="c", subcore_axis_name="s",
                                  num_cores=NC, num_subcores=NS)


def _sc_dispatch(pos_all, m, *, nrows, t, dim, nslots):
    nw = NC * NS
    spt = nslots // nw
    nch = 4
    ch = spt // nch

    @functools.partial(
        pl.kernel,
        out_type=jax.ShapeDtypeStruct((nrows, dim), jnp.float32),
        mesh=_sc_mesh(),
        scratch_types=([pltpu.VMEM((ch,), jnp.int32) for _ in range(nch)]
                       + [pltpu.VMEM((ch, dim), jnp.float32),
                          pltpu.VMEM((ch, dim), jnp.float32),
                          pltpu.SemaphoreType.DMA,
                          pltpu.SemaphoreType.DMA]),
        compiler_params=pltpu.CompilerParams(needs_layout_passes=False),
    )
    def _dispatch(pos_hbm, m_hbm, xg_hbm, *refs):
        idxs = refs[:nch]
        bufs = refs[nch:nch + 2]
        sems = refs[nch + 2:]
        wid = lax.axis_index("s") * NC + lax.axis_index("c")
        base = wid * spt
        tok0 = base % t
        cps = []
        for ci in range(nch):
            buf = bufs[ci % 2]
            if ci >= 2:
                cps[ci - 2].wait()
            pltpu.sync_copy(pos_hbm.at[pl.ds(base + ci * ch, ch)], idxs[ci])
            pltpu.sync_copy(m_hbm.at[pl.ds(tok0 + ci * ch, ch)], buf)
            cps.append(pltpu.async_copy(buf, xg_hbm.at[idxs[ci]],
                                        sems[ci % 2]))
        cps[nch - 2].wait()
        cps[nch - 1].wait()

    return _dispatch(pos_all, m)


def _sc_combine(pos_all, y, *, dim, nslots):
    nw = NC * NS
    spt = nslots // nw
    nch = 4
    ch = spt // nch

    @functools.partial(
        pl.kernel,
        out_type=jax.ShapeDtypeStruct((nslots, dim), jnp.float32),
        mesh=_sc_mesh(),
        scratch_types=[pltpu.VMEM((spt,), jnp.int32),
                       pltpu.VMEM((ch, dim), jnp.float32),
                       pltpu.VMEM((ch, dim), jnp.float32),
                       pltpu.SemaphoreType.DMA,
                       pltpu.SemaphoreType.DMA],
        compiler_params=pltpu.CompilerParams(needs_layout_passes=False),
    )
    def _combine(pos_hbm, y_hbm, yg_hbm, idx_v, buf_a, buf_b, sem_a, sem_b):
        bufs = (buf_a, buf_b)
        sems = (sem_a, sem_b)
        wid = lax.axis_index("s") * NC + lax.axis_index("c")
        base = wid * spt
        pltpu.sync_copy(pos_hbm.at[pl.ds(base, spt)], idx_v)
        cps = []
        for ci in range(nch):
            buf = bufs[ci % 2]
            if ci >= 2:
                cps[ci - 2].wait()
                pltpu.sync_copy(buf, yg_hbm.at[pl.ds(base + (ci - 2) * ch, ch)])
            cps.append(pltpu.async_copy(
                y_hbm.at[idx_v.at[pl.ds(ci * ch, ch)]], buf, sems[ci % 2]))
        for ci in range(nch - 2, nch):
            cps[ci].wait()
            pltpu.sync_copy(bufs[ci % 2], yg_hbm.at[pl.ds(base + ci * ch, ch)])

    return _combine(pos_all, y)


def _final_body(x2_ref, ya_ref, yb_ref, wa_ref, wb_ref, o_ref):
    wa = jnp.reshape(wa_ref[...], (-1, 1))
    wb = jnp.reshape(wb_ref[...], (-1, 1))
    o_ref[...] = x2_ref[...] + wa * ya_ref[...] + wb * yb_ref[...]


def kernel(x, attn_norm_w, Wq, Wk, Wv, Wo, q_norm_w, k_norm_w, ffn_norm_w,
           Wi, Wg, Woe, router_w, cos, sin):
    b, t, dim = x.shape
    nq = Wq.shape[1] // cos.shape[1]
    nkv = Wk.shape[1] // cos.shape[1]
    hd = cos.shape[1]
    ne, _, hid = Wi.shape
    blk = min(BLK, t)
    nt = t // blk
    nslots = 2 * t
    nblk = nslots // BLKF + ne
    nrows = nblk * BLKF

    x2d = x.reshape(t, dim)
    bf = jnp.bfloat16
    f32 = jnp.float32
    wi_b, wg_b, woe_b = Wi, Wg, Woe

    p64 = _rot_perm(hd)
    pq = jnp.asarray(np.kron(np.eye(nq, dtype=np.float32), p64), bf)
    pk = jnp.asarray(np.kron(np.eye(nkv, dtype=np.float32), p64), bf)
    hq = jnp.asarray(np.kron(np.eye(nq, dtype=np.float32),
                             np.ones((hd, 1), np.float32)))
    hk = jnp.asarray(np.kron(np.eye(nkv, dtype=np.float32),
                             np.ones((hd, 1), np.float32)))
    qnw = jnp.tile(q_norm_w, (nq,)).reshape(1, nq * hd)
    knw = jnp.tile(k_norm_w, (nkv,)).reshape(1, nkv * hd)
    anw = attn_norm_w.reshape(1, dim)
    fnw = ffn_norm_w.reshape(1, dim)

    dq, dkv = nq * hd, nkv * hd
    rep = nq // nkv

    full = lambda shape: pl.BlockSpec(shape, lambda *_: (0,) * len(shape))
    rowblk = lambda w: pl.BlockSpec((blk, w), lambda i: (i, 0))

    q, k, v = pl.pallas_call(
        functools.partial(_prelude_body, hd=hd, nq=nq, nkv=nkv),
        grid=(nt,),
        in_specs=[
            rowblk(dim), full((1, dim)), full((dim, dq)), full((dim, dkv)),
            full((dim, dkv)), full((1, dq)), full((1, dkv)),
            rowblk(hd), rowblk(hd),
            full((dq, dq)), full((dkv, dkv)),
            full((dq, nq)), full((nq, dq)), full((dkv, nkv)), full((nkv, dkv)),
        ],
        out_specs=[rowblk(dq), rowblk(dkv), rowblk(dkv)],
        out_shape=[
            jax.ShapeDtypeStruct((t, dq), bf),
            jax.ShapeDtypeStruct((t, dkv), bf),
            jax.ShapeDtypeStruct((t, dkv), bf),
        ],
    )(x2d, anw, Wq, Wk, Wv, qnw, knw, cos, sin,
      pq, pk, hq, hq.T, hk, hk.T)

    x2, m, wa, wb, posa, posb, be = pl.pallas_call(
        functools.partial(_attnpost_body, hd=hd, nq=nq, rep=rep, ne=ne,
                          nblk=nblk, nt=nt, blk=blk),
        grid=(nt,),
        in_specs=[rowblk(dq), full((t, dkv)), full((t, dkv)),
                  full((dq, dim)), rowblk(dim), full((1, dim)),
                  full((dim, ne))],
        out_specs=[rowblk(dim), rowblk(dim),
                   pl.BlockSpec((blk,), lambda i: (i,)),
                   pl.BlockSpec((blk,), lambda i: (i,)),
                   pl.BlockSpec((t,), lambda i: (0,)),
                   pl.BlockSpec((t,), lambda i: (0,)),
                   pl.BlockSpec((nblk,), lambda i: (0,))],
        out_shape=[
            jax.ShapeDtypeStruct((t, dim), f32),
            jax.ShapeDtypeStruct((t, dim), f32),
            jax.ShapeDtypeStruct((t,), f32),
            jax.ShapeDtypeStruct((t,), f32),
            jax.ShapeDtypeStruct((t,), jnp.int32),
            jax.ShapeDtypeStruct((t,), jnp.int32),
            jax.ShapeDtypeStruct((nblk,), jnp.int32),
        ],
        scratch_shapes=[pltpu.VMEM((t, ne), f32), pltpu.VMEM((t, ne), f32)],
    )(q, k, v, Wo, x2d, fnw, router_w)

    pos_all = jnp.concatenate([posa, posb])
    xg = _sc_dispatch(pos_all, m, nrows=nrows, t=t, dim=dim, nslots=nslots)

    y = pl.pallas_call(
        _gffn_body,
        grid_spec=pltpu.PrefetchScalarGridSpec(
            num_scalar_prefetch=1,
            grid=(nblk,),
            in_specs=[
                pl.BlockSpec((BLKF, dim), lambda bi, be_s: (bi, 0)),
                pl.BlockSpec((1, dim, hid),
                             lambda bi, be_s: (jnp.maximum(be_s[bi], 0), 0, 0)),
                pl.BlockSpec((1, dim, hid),
                             lambda bi, be_s: (jnp.maximum(be_s[bi], 0), 0, 0)),
                pl.BlockSpec((1, hid, dim),
                             lambda bi, be_s: (jnp.maximum(be_s[bi], 0), 0, 0)),
            ],
            out_specs=pl.BlockSpec((BLKF, dim), lambda bi, be_s: (bi, 0)),
        ),
        out_shape=jax.ShapeDtypeStruct((nrows, dim), f32),
        compiler_params=pltpu.CompilerParams(
            dimension_semantics=("arbitrary",)),
    )(be, xg, wg_b, wi_b, woe_b)

    yg = _sc_combine(pos_all, y, dim=dim, nslots=nslots)
    ya, yb = yg[:t], yg[t:]

    out = pl.pallas_call(
        _final_body,
        grid=(nt,),
        in_specs=[rowblk(dim), rowblk(dim), rowblk(dim),
                  pl.BlockSpec((blk,), lambda i: (i,)),
                  pl.BlockSpec((blk,), lambda i: (i,))],
        out_specs=rowblk(dim),
        out_shape=jax.ShapeDtypeStruct((t, dim), f32),
    )(x2, ya, yb, wa, wb)

    return out.reshape(b, t, dim)
